# baseline (speedup 1.0000x reference)
import functools

import jax
import jax.numpy as jnp
from jax import lax
from jax.experimental import pallas as pl
from jax.experimental.pallas import tpu as pltpu
from jax.experimental.pallas import tpu_sc as plsc

_K = 16
_C = 128
_D = 64
_NB0 = 512
_BSQ = 128
_BS3 = 128


def _dot(x, w):
    return lax.dot_general(x, w, (((1,), (1,)), ((), ())),
                           preferred_element_type=jnp.float32)


def _precompute_body(f1_ref, f2_ref, p1_ref, p2_ref, w0a_ref, w0b_ref,
                     b0_ref, a1_ref, t2_ref, q1_ref, k1_ref, k2_ref):
    f1 = f1_ref[0]
    f2 = f2_ref[0]
    a1 = lax.dot_general(f1, w0a_ref[...], (((0,), (1,)), ((), ())),
                         preferred_element_type=jnp.float32)
    a2 = lax.dot_general(f2, w0b_ref[...], (((0,), (1,)), ((), ())),
                         preferred_element_type=jnp.float32)
    a1_ref[0] = a1 + b0_ref[...]
    t2_ref[0] = a2
    p1 = p1_ref[0]
    p2 = p2_ref[0]
    nb = p1.shape[0]
    one = jnp.ones((nb, 1), jnp.float32)
    zero = jnp.zeros((nb, 3), jnp.float32)
    p1sq = jnp.sum(p1 * p1, axis=1, keepdims=True)
    p2sq = jnp.sum(p2 * p2, axis=1, keepdims=True)
    q1_ref[0] = jnp.concatenate([-2.0 * p1, one, p1sq, zero], axis=1)
    k1_ref[0] = jnp.concatenate([p1, p1sq, one, zero], axis=1)
    k2_ref[0] = jnp.concatenate([p2, p2sq, one, zero], axis=1)


def _precompute(feature1, feature2, p1T, p2T, w0a, w0b, b0row):
    B, D, N = feature1.shape
    grid = (B, N // _NB0)
    aug = jax.ShapeDtypeStruct((B, N, 8), jnp.float32)
    return pl.pallas_call(
        _precompute_body,
        grid=grid,
        in_specs=[
            pl.BlockSpec((1, D, _NB0), lambda b, i: (b, 0, i)),
            pl.BlockSpec((1, D, _NB0), lambda b, i: (b, 0, i)),
            pl.BlockSpec((1, _NB0, 3), lambda b, i: (b, i, 0)),
            pl.BlockSpec((1, _NB0, 3), lambda b, i: (b, i, 0)),
            pl.BlockSpec((_C, D), lambda b, i: (0, 0)),
            pl.BlockSpec((_C, D), lambda b, i: (0, 0)),
            pl.BlockSpec((1, _C), lambda b, i: (0, 0)),
        ],
        out_specs=[
            pl.BlockSpec((1, _NB0, _C), lambda b, i: (b, i, 0)),
            pl.BlockSpec((1, _NB0, _C), lambda b, i: (b, i, 0)),
            pl.BlockSpec((1, _NB0, 8), lambda b, i: (b, i, 0)),
            pl.BlockSpec((1, _NB0, 8), lambda b, i: (b, i, 0)),
            pl.BlockSpec((1, _NB0, 8), lambda b, i: (b, i, 0)),
        ],
        out_shape=[
            jax.ShapeDtypeStruct((B, N, _C), jnp.float32),
            jax.ShapeDtypeStruct((B, N, _C), jnp.float32),
            aug, aug, aug,
        ],
    )(feature1, feature2, p1T, p2T, w0a, w0b, b0row)


_S = 4


def _ce(a, b):
    return jnp.minimum(a, b), jnp.maximum(a, b)


def _sort4(a, b, c, d):
    a, b = _ce(a, b)
    c, d = _ce(c, d)
    a, c = _ce(a, c)
    b, d = _ce(b, d)
    b, c = _ce(b, c)
    return [a, b, c, d]


def _merge4(x, y):
    c0 = jnp.minimum(x[0], y[3])
    c1 = jnp.minimum(x[1], y[2])
    c2 = jnp.minimum(x[2], y[1])
    c3 = jnp.minimum(x[3], y[0])
    c0, c2 = _ce(c0, c2)
    c1, c3 = _ce(c1, c3)
    c0, c1 = _ce(c0, c1)
    c2, c3 = _ce(c2, c3)
    return [c0, c1, c2, c3]


def _knn_body(q_ref, k_ref, idx_ref, *, n_keys):
    qa = q_ref[0]
    ka = k_ref[0]
    bs = qa.shape[0]
    ng = n_keys // 128
    kq = lax.dot_general(ka[:, :3], qa[:, :3], (((1,), (1,)), ((), ())),
                         preferred_element_type=jnp.float32)
    knb = lax.dot_general(ka[:, 3:4], qa[:, 3:4], (((1,), (1,)), ((), ())),
                          precision=lax.Precision.HIGHEST,
                          preferred_element_type=jnp.float32)
    e4 = (lax.broadcasted_iota(jnp.int32, (1, 8), 1) == 4).astype(jnp.float32)
    qn = lax.dot_general(e4, qa, (((1,), (1,)), ((), ())),
                         preferred_element_type=jnp.float32)
    distT = kq + knb + qn
    keys3 = (lax.bitcast_convert_type(distT, jnp.int32).reshape(ng, 128, bs)
             & jnp.int32(-ng)) | lax.broadcasted_iota(jnp.int32,
                                                      (ng, 128, bs), 0)
    maxi = jnp.int32(2 ** 31 - 1)
    big = jnp.int32(2 ** 30)
    quads = [_sort4(keys3[4 * i], keys3[4 * i + 1],
                    keys3[4 * i + 2], keys3[4 * i + 3])
             for i in range(ng // 4)]
    while len(quads) > 1:
        quads = [_merge4(quads[2 * i], quads[2 * i + 1])
                 for i in range(len(quads) // 2)]
    stack = quads[0]
    s_iota = lax.broadcasted_iota(jnp.int32, (128, bs), 0)
    colcur = stack[0]
    cnt = jnp.zeros((128, bs), jnp.int32)
    rows = []
    for _ in range(_K):
        m = jnp.min(colcur, axis=0, keepdims=True)
        sel = colcur == m
        sstar = jnp.min(jnp.where(sel, s_iota, big),
                        axis=0, keepdims=True)
        cstar = m & jnp.int32(ng - 1)
        rows.append(cstar * 128 + sstar)
        hit = s_iota == sstar
        cnt = cnt + jnp.where(hit, 1, 0)
        refill = jnp.full((128, bs), maxi, jnp.int32)
        for s in range(1, _S):
            refill = jnp.where(cnt == s, stack[s], refill)
        colcur = jnp.where(hit, refill, colcur)
    idx = jnp.concatenate(rows, axis=0)
    idx_ref[0] = idx + pl.program_id(0) * n_keys


def _knn(q_aug, k_aug):
    B, N1, _ = q_aug.shape
    N2 = k_aug.shape[1]
    return pl.pallas_call(
        functools.partial(_knn_body, n_keys=N2),
        grid=(B, N1 // _BSQ),
        in_specs=[
            pl.BlockSpec((1, _BSQ, 8), lambda b, i: (b, i, 0)),
            pl.BlockSpec((1, N2, 8), lambda b, i: (b, 0, 0)),
        ],
        out_specs=pl.BlockSpec((1, _K, _BSQ), lambda b, i: (b, 0, i)),
        out_shape=jax.ShapeDtypeStruct((B, _K, N1), jnp.int32),
    )(q_aug, k_aug)


def _sc_gather(table, xrow, yrow, zrow, idx):
    total = idx.shape[0]
    n_chunks = total // 128
    idx2d = idx.reshape(n_chunks, 128)
    per_w = n_chunks // 32
    npts = xrow.shape[0]
    mesh = plsc.VectorSubcoreMesh(core_axis_name="c", subcore_axis_name="s")

    @functools.partial(
        pl.kernel, mesh=mesh,
        compiler_params=pltpu.CompilerParams(needs_layout_passes=False),
        out_type=[
            jax.ShapeDtypeStruct((total, _C), jnp.float32),
            jax.ShapeDtypeStruct((n_chunks, 4, 128), jnp.float32),
        ],
        scratch_types=[
            pltpu.VMEM((128,), jnp.int32),
            pltpu.VMEM((128, _C), jnp.float32),
            pltpu.VMEM((4, 128), jnp.float32),
            pltpu.VMEM((npts,), jnp.float32),
            pltpu.VMEM((npts,), jnp.float32),
            pltpu.VMEM((npts,), jnp.float32),
            pltpu.SemaphoreType.DMA,
        ],
    )
    def gk(table_hbm, x_hbm, y_hbm, z_hbm, idx_hbm, out_hbm, xyz_hbm,
           idxv, rows, xyzbuf, xv, yv, zv, sem):
        wid = lax.axis_index("s") * 2 + lax.axis_index("c")
        pltpu.sync_copy(x_hbm, xv)
        pltpu.sync_copy(y_hbm, yv)
        pltpu.sync_copy(z_hbm, zv)

        def body(c, carry):
            row = wid * per_w + c
            pltpu.sync_copy(idx_hbm.at[row], idxv)
            pltpu.async_copy(table_hbm.at[idxv], rows, sem).wait()
            for g in range(8):
                iv = idxv[pl.ds(g * 16, 16)]
                xyzbuf[0, pl.ds(g * 16, 16)] = plsc.load_gather(xv, [iv])
                xyzbuf[1, pl.ds(g * 16, 16)] = plsc.load_gather(yv, [iv])
                xyzbuf[2, pl.ds(g * 16, 16)] = plsc.load_gather(zv, [iv])
            pltpu.sync_copy(rows, out_hbm.at[pl.ds(row * 128, 128)])
            pltpu.sync_copy(xyzbuf, xyz_hbm.at[row])
            return carry

        lax.fori_loop(0, per_w, body, 0)

    return gk(table, xrow, yrow, zrow, idx2d)


def _phase3_body(g_ref, gx_ref, a1_ref, p1_ref, m1_ref, c0_ref, w1_ref,
                 b1_ref, v1_ref, c1_ref, v2_ref, c2_ref, out_ref):
    bs = p1_ref.shape[1]
    r = bs * _K
    a2 = g_ref[0].reshape(r, _C)
    xyzj = gx_ref[0].reshape(r, 4)[:, :3]
    p1 = p1_ref[0]
    p1r = jnp.broadcast_to(p1[None, :, :], (_K, bs, 3)).reshape(r, 3)
    d = xyzj - p1r
    t = _dot(d, m1_ref[...])
    h = jnp.maximum(t[:, :8] + c0_ref[...], 0.0)
    dirproj = t[:, 8:8 + _C]
    a1 = a1_ref[0]
    a1r = jnp.broadcast_to(a1[None, :, :], (_K, bs, _C)).reshape(r, _C)
    x = a1r + a2 + dirproj
    x = jnp.where(x >= 0.0, x, 0.1 * x)
    y = _dot(x, w1_ref[...]) + b1_ref[...]
    y = jnp.where(y >= 0.0, y, 0.1 * y)
    h = jnp.maximum(_dot(h, v1_ref[...]) + c1_ref[...], 0.0)
    w = jnp.maximum(_dot(h, v2_ref[...]) + c2_ref[...], 0.0)
    out_ref[0] = jnp.sum((w * y).reshape(_K, bs, _C), axis=0)


def _phase3(g1, g1x, a1rows, p1T, m1, c0, w1, b1, v1, c1, v2, c2):
    B, N1, _ = p1T.shape
    return pl.pallas_call(
        _phase3_body,
        grid=(B, N1 // _BS3),
        in_specs=[
            pl.BlockSpec((1, _K, _BS3, _C), lambda b, i: (b, 0, i, 0)),
            pl.BlockSpec((1, _K, _BS3, 4), lambda b, i: (b, 0, i, 0)),
            pl.BlockSpec((1, _BS3, _C), lambda b, i: (b, i, 0)),
            pl.BlockSpec((1, _BS3, 3), lambda b, i: (b, i, 0)),
            pl.BlockSpec((8 + _C, 3), lambda b, i: (0, 0)),
            pl.BlockSpec((1, 8), lambda b, i: (0, 0)),
            pl.BlockSpec((_C, _C), lambda b, i: (0, 0)),
            pl.BlockSpec((1, _C), lambda b, i: (0, 0)),
            pl.BlockSpec((8, 8), lambda b, i: (0, 0)),
            pl.BlockSpec((1, 8), lambda b, i: (0, 0)),
            pl.BlockSpec((_C, 8), lambda b, i: (0, 0)),
            pl.BlockSpec((1, _C), lambda b, i: (0, 0)),
        ],
        out_specs=pl.BlockSpec((1, _BS3, _C), lambda b, i: (b, i, 0)),
        out_shape=jax.ShapeDtypeStruct((B, N1, _C), jnp.float32),
    )(g1, g1x, a1rows, p1T, m1, c0, w1, b1, v1, c1, v2, c2)


def _phase5_body(g_ref, gx_ref, p1_ref, v0_ref, c0_ref, v1_ref, c1_ref,
                 v2_ref, c2_ref, out_ref):
    bs = p1_ref.shape[1]
    r = bs * _K
    nfj = g_ref[0].reshape(r, _C)
    xyzj = gx_ref[0].reshape(r, 4)[:, :3]
    p1 = p1_ref[0]
    p1r = jnp.broadcast_to(p1[None, :, :], (_K, bs, 3)).reshape(r, 3)
    d = xyzj - p1r
    h = jnp.maximum(_dot(d, v0_ref[...]) + c0_ref[...], 0.0)
    h = jnp.maximum(_dot(h, v1_ref[...]) + c1_ref[...], 0.0)
    w = jnp.maximum(_dot(h, v2_ref[...]) + c2_ref[...], 0.0)
    out_ref[0] = jnp.sum((w * nfj).reshape(_K, bs, _C), axis=0)


def _phase5(g2, g2x, p1T, v0, c0, v1, c1, v2, c2):
    B, N1, _ = p1T.shape
    return pl.pallas_call(
        _phase5_body,
        grid=(B, N1 // _BS3),
        in_specs=[
            pl.BlockSpec((1, _K, _BS3, _C), lambda b, i: (b, 0, i, 0)),
            pl.BlockSpec((1, _K, _BS3, 4), lambda b, i: (b, 0, i, 0)),
            pl.BlockSpec((1, _BS3, 3), lambda b, i: (b, i, 0)),
            pl.BlockSpec((8, 3), lambda b, i: (0, 0)),
            pl.BlockSpec((1, 8), lambda b, i: (0, 0)),
            pl.BlockSpec((8, 8), lambda b, i: (0, 0)),
            pl.BlockSpec((1, 8), lambda b, i: (0, 0)),
            pl.BlockSpec((_C, 8), lambda b, i: (0, 0)),
            pl.BlockSpec((1, _C), lambda b, i: (0, 0)),
        ],
        out_specs=pl.BlockSpec((1, _BS3, _C), lambda b, i: (b, i, 0)),
        out_shape=jax.ShapeDtypeStruct((B, N1, _C), jnp.float32),
    )(g2, g2x, p1T, v0, c0, v1, c1, v2, c2)


def kernel(pc1, pc2, feature1, feature2, mlp_W0, mlp_b0, mlp_W1, mlp_b1,
           wn1_W0, wn1_b0, wn1_W1, wn1_b1, wn1_W2, wn1_b2,
           wn2_W0, wn2_b0, wn2_W1, wn2_b1, wn2_W2, wn2_b2):
    B, _, N1 = pc1.shape
    N2 = pc2.shape[2]
    p1T = jnp.transpose(pc1, (0, 2, 1))
    w0a = mlp_W0[:, :_D]
    w0b = mlp_W0[:, _D:2 * _D]
    m1 = jnp.concatenate([wn1_W0, mlp_W0[:, 2 * _D:]], axis=0)

    p2T = jnp.transpose(pc2, (0, 2, 1))
    a1rows, t2, q1aug, k1aug, k2aug = _precompute(
        feature1, feature2, p1T, p2T, w0a, w0b, mlp_b0[None])
    idx1 = _knn(q1aug, k2aug)
    idx2 = _knn(q1aug, k1aug)
    total = B * N1 * _K

    p2rows = jnp.transpose(pc2, (1, 0, 2)).reshape(3, B * N2)
    p1rows = jnp.transpose(pc1, (1, 0, 2)).reshape(3, B * N1)

    g1, g1xc = _sc_gather(t2.reshape(B * N2, _C),
                          p2rows[0], p2rows[1], p2rows[2], idx1.reshape(-1))
    g1x = jnp.transpose(g1xc, (0, 2, 1)).reshape(total, 4)
    nf = _phase3(g1.reshape(B, _K, N1, _C), g1x.reshape(B, _K, N1, 4),
                 a1rows, p1T, m1,
                 wn1_b0[None], mlp_W1, mlp_b1[None],
                 wn1_W1, wn1_b1[None], wn1_W2, wn1_b2[None])

    g2, g2xc = _sc_gather(nf.reshape(B * N1, _C),
                          p1rows[0], p1rows[1], p1rows[2], idx2.reshape(-1))
    g2x = jnp.transpose(g2xc, (0, 2, 1)).reshape(total, 4)
    out_rows = _phase5(g2.reshape(B, _K, N1, _C), g2x.reshape(B, _K, N1, 4),
                       p1T,
                       wn2_W0, wn2_b0[None], wn2_W1, wn2_b1[None],
                       wn2_W2, wn2_b2[None])
    return jnp.transpose(out_rows, (0, 2, 1))


# speedup vs baseline: 1.3915x; 1.3915x over previous
import functools

import jax
import jax.numpy as jnp
from jax import lax
from jax.experimental import pallas as pl
from jax.experimental.pallas import tpu as pltpu
from jax.experimental.pallas import tpu_sc as plsc

_K = 16
_C = 128
_D = 64
_NB0 = 512
_BSQ = 128
_BS3 = 128


def _dot(x, w):
    return lax.dot_general(x, w, (((1,), (1,)), ((), ())),
                           preferred_element_type=jnp.float32)


def _precompute_body(f1_ref, f2_ref, p1_ref, p2_ref, w0a_ref, w0b_ref,
                     b0_ref, a1_ref, t2_ref, q1_ref, k1_ref, k2_ref):
    f1 = f1_ref[0]
    f2 = f2_ref[0]
    a1 = lax.dot_general(f1, w0a_ref[...], (((0,), (1,)), ((), ())),
                         preferred_element_type=jnp.float32)
    a2 = lax.dot_general(f2, w0b_ref[...], (((0,), (1,)), ((), ())),
                         preferred_element_type=jnp.float32)
    a1_ref[0] = a1 + b0_ref[...]
    t2_ref[0] = a2
    p1 = p1_ref[0]
    p2 = p2_ref[0]
    nb = p1.shape[0]
    one = jnp.ones((nb, 1), jnp.float32)
    zero = jnp.zeros((nb, 3), jnp.float32)
    p1sq = jnp.sum(p1 * p1, axis=1, keepdims=True)
    p2sq = jnp.sum(p2 * p2, axis=1, keepdims=True)
    q1_ref[0] = jnp.concatenate([-2.0 * p1, one, p1sq, zero], axis=1)
    k1_ref[0] = jnp.concatenate([p1, p1sq, one, zero], axis=1)
    k2_ref[0] = jnp.concatenate([p2, p2sq, one, zero], axis=1)


def _precompute(feature1, feature2, p1T, p2T, w0a, w0b, b0row):
    B, D, N = feature1.shape
    grid = (B, N // _NB0)
    aug = jax.ShapeDtypeStruct((B, N, 8), jnp.float32)
    return pl.pallas_call(
        _precompute_body,
        grid=grid,
        in_specs=[
            pl.BlockSpec((1, D, _NB0), lambda b, i: (b, 0, i)),
            pl.BlockSpec((1, D, _NB0), lambda b, i: (b, 0, i)),
            pl.BlockSpec((1, _NB0, 3), lambda b, i: (b, i, 0)),
            pl.BlockSpec((1, _NB0, 3), lambda b, i: (b, i, 0)),
            pl.BlockSpec((_C, D), lambda b, i: (0, 0)),
            pl.BlockSpec((_C, D), lambda b, i: (0, 0)),
            pl.BlockSpec((1, _C), lambda b, i: (0, 0)),
        ],
        out_specs=[
            pl.BlockSpec((1, _NB0, _C), lambda b, i: (b, i, 0)),
            pl.BlockSpec((1, _NB0, _C), lambda b, i: (b, i, 0)),
            pl.BlockSpec((1, _NB0, 8), lambda b, i: (b, i, 0)),
            pl.BlockSpec((1, _NB0, 8), lambda b, i: (b, i, 0)),
            pl.BlockSpec((1, _NB0, 8), lambda b, i: (b, i, 0)),
        ],
        out_shape=[
            jax.ShapeDtypeStruct((B, N, _C), jnp.float32),
            jax.ShapeDtypeStruct((B, N, _C), jnp.float32),
            aug, aug, aug,
        ],
    )(feature1, feature2, p1T, p2T, w0a, w0b, b0row)


_S = 4


def _ce(a, b):
    return jnp.minimum(a, b), jnp.maximum(a, b)


def _sort4(a, b, c, d):
    a, b = _ce(a, b)
    c, d = _ce(c, d)
    a, c = _ce(a, c)
    b, d = _ce(b, d)
    b, c = _ce(b, c)
    return [a, b, c, d]


def _merge4(x, y):
    c0 = jnp.minimum(x[0], y[3])
    c1 = jnp.minimum(x[1], y[2])
    c2 = jnp.minimum(x[2], y[1])
    c3 = jnp.minimum(x[3], y[0])
    c0, c2 = _ce(c0, c2)
    c1, c3 = _ce(c1, c3)
    c0, c1 = _ce(c0, c1)
    c2, c3 = _ce(c2, c3)
    return [c0, c1, c2, c3]


def _knn_body(q_ref, k_ref, idx_ref, *, n_keys):
    qa = q_ref[0]
    ka = k_ref[0]
    bs = qa.shape[0]
    ng = n_keys // 128
    kq = lax.dot_general(ka[:, :3], qa[:, :3], (((1,), (1,)), ((), ())),
                         preferred_element_type=jnp.float32)
    e4 = (lax.broadcasted_iota(jnp.int32, (1, 8), 1) == 4).astype(jnp.float32)
    qn = lax.dot_general(e4, qa, (((1,), (1,)), ((), ())),
                         preferred_element_type=jnp.float32)
    kn = ka[:, 3:4]
    distT = kq + kn + qn
    keys3 = (lax.bitcast_convert_type(distT, jnp.int32).reshape(ng, 128, bs)
             & jnp.int32(-ng)) | lax.broadcasted_iota(jnp.int32,
                                                      (ng, 128, bs), 0)
    maxi = jnp.int32(2 ** 31 - 1)
    big = jnp.int32(2 ** 30)
    quads = [_sort4(keys3[4 * i], keys3[4 * i + 1],
                    keys3[4 * i + 2], keys3[4 * i + 3])
             for i in range(ng // 4)]
    while len(quads) > 1:
        quads = [_merge4(quads[2 * i], quads[2 * i + 1])
                 for i in range(len(quads) // 2)]
    stack = quads[0]
    s_iota = lax.broadcasted_iota(jnp.int32, (128, bs), 0)
    colcur = stack[0]
    cnt = jnp.zeros((128, bs), jnp.int32)
    rows = []
    for _ in range(_K):
        m = jnp.min(colcur, axis=0, keepdims=True)
        sel = colcur == m
        sstar = jnp.min(jnp.where(sel, s_iota, big),
                        axis=0, keepdims=True)
        cstar = m & jnp.int32(ng - 1)
        rows.append(cstar * 128 + sstar)
        hit = s_iota == sstar
        cnt = cnt + jnp.where(hit, 1, 0)
        refill = jnp.full((128, bs), maxi, jnp.int32)
        for s in range(1, _S):
            refill = jnp.where(cnt == s, stack[s], refill)
        colcur = jnp.where(hit, refill, colcur)
    idx = jnp.concatenate(rows, axis=0)
    idx_ref[0] = idx + pl.program_id(0) * n_keys


def _knn(q_aug, k_aug):
    B, N1, _ = q_aug.shape
    N2 = k_aug.shape[1]
    return pl.pallas_call(
        functools.partial(_knn_body, n_keys=N2),
        grid=(B, N1 // _BSQ),
        in_specs=[
            pl.BlockSpec((1, _BSQ, 8), lambda b, i: (b, i, 0)),
            pl.BlockSpec((1, N2, 8), lambda b, i: (b, 0, 0)),
        ],
        out_specs=pl.BlockSpec((1, _K, _BSQ), lambda b, i: (b, 0, i)),
        out_shape=jax.ShapeDtypeStruct((B, _K, N1), jnp.int32),
    )(q_aug, k_aug)


def _sc_gather(table, xrow, yrow, zrow, idx):
    total = idx.shape[0]
    n_chunks = total // 128
    idx2d = idx.reshape(n_chunks, 128)
    per_w = n_chunks // 32
    npts = xrow.shape[0]
    mesh = plsc.VectorSubcoreMesh(core_axis_name="c", subcore_axis_name="s")

    @functools.partial(
        pl.kernel, mesh=mesh,
        compiler_params=pltpu.CompilerParams(needs_layout_passes=False),
        out_type=[
            jax.ShapeDtypeStruct((total, _C), jnp.float32),
            jax.ShapeDtypeStruct((n_chunks, 4, 128), jnp.float32),
        ],
        scratch_types=[
            pltpu.VMEM((128,), jnp.int32),
            pltpu.VMEM((128, _C), jnp.float32),
            pltpu.VMEM((4, 128), jnp.float32),
            pltpu.VMEM((npts,), jnp.float32),
            pltpu.VMEM((npts,), jnp.float32),
            pltpu.VMEM((npts,), jnp.float32),
            pltpu.SemaphoreType.DMA,
        ],
    )
    def gk(table_hbm, x_hbm, y_hbm, z_hbm, idx_hbm, out_hbm, xyz_hbm,
           idxv, rows, xyzbuf, xv, yv, zv, sem):
        wid = lax.axis_index("s") * 2 + lax.axis_index("c")
        pltpu.sync_copy(x_hbm, xv)
        pltpu.sync_copy(y_hbm, yv)
        pltpu.sync_copy(z_hbm, zv)

        def body(c, carry):
            row = wid * per_w + c
            pltpu.sync_copy(idx_hbm.at[row], idxv)
            pltpu.async_copy(table_hbm.at[idxv], rows, sem).wait()
            for g in range(8):
                iv = idxv[pl.ds(g * 16, 16)]
                xyzbuf[0, pl.ds(g * 16, 16)] = plsc.load_gather(xv, [iv])
                xyzbuf[1, pl.ds(g * 16, 16)] = plsc.load_gather(yv, [iv])
                xyzbuf[2, pl.ds(g * 16, 16)] = plsc.load_gather(zv, [iv])
            pltpu.sync_copy(rows, out_hbm.at[pl.ds(row * 128, 128)])
            pltpu.sync_copy(xyzbuf, xyz_hbm.at[row])
            return carry

        lax.fori_loop(0, per_w, body, 0)

    return gk(table, xrow, yrow, zrow, idx2d)


def _phase3_body(g_ref, gx_ref, a1_ref, p1_ref, m1_ref, c0_ref, w1_ref,
                 b1_ref, v1_ref, c1_ref, v2_ref, c2_ref, out_ref):
    bs = p1_ref.shape[1]
    r = bs * _K
    a2 = g_ref[0].reshape(r, _C)
    xyzj = gx_ref[0].reshape(r, 4)[:, :3]
    p1 = p1_ref[0]
    p1r = jnp.broadcast_to(p1[None, :, :], (_K, bs, 3)).reshape(r, 3)
    d = xyzj - p1r
    t = _dot(d, m1_ref[...])
    h = jnp.maximum(t[:, :8] + c0_ref[...], 0.0)
    dirproj = t[:, 8:8 + _C]
    a1 = a1_ref[0]
    a1r = jnp.broadcast_to(a1[None, :, :], (_K, bs, _C)).reshape(r, _C)
    x = a1r + a2 + dirproj
    x = jnp.where(x >= 0.0, x, 0.1 * x)
    y = _dot(x, w1_ref[...]) + b1_ref[...]
    y = jnp.where(y >= 0.0, y, 0.1 * y)
    h = jnp.maximum(_dot(h, v1_ref[...]) + c1_ref[...], 0.0)
    w = jnp.maximum(_dot(h, v2_ref[...]) + c2_ref[...], 0.0)
    out_ref[0] = jnp.sum((w * y).reshape(_K, bs, _C), axis=0)


def _phase3(g1, g1x, a1rows, p1T, m1, c0, w1, b1, v1, c1, v2, c2):
    B, N1, _ = p1T.shape
    return pl.pallas_call(
        _phase3_body,
        grid=(B, N1 // _BS3),
        in_specs=[
            pl.BlockSpec((1, _K, _BS3, _C), lambda b, i: (b, 0, i, 0)),
            pl.BlockSpec((1, _K, _BS3, 4), lambda b, i: (b, 0, i, 0)),
            pl.BlockSpec((1, _BS3, _C), lambda b, i: (b, i, 0)),
            pl.BlockSpec((1, _BS3, 3), lambda b, i: (b, i, 0)),
            pl.BlockSpec((8 + _C, 3), lambda b, i: (0, 0)),
            pl.BlockSpec((1, 8), lambda b, i: (0, 0)),
            pl.BlockSpec((_C, _C), lambda b, i: (0, 0)),
            pl.BlockSpec((1, _C), lambda b, i: (0, 0)),
            pl.BlockSpec((8, 8), lambda b, i: (0, 0)),
            pl.BlockSpec((1, 8), lambda b, i: (0, 0)),
            pl.BlockSpec((_C, 8), lambda b, i: (0, 0)),
            pl.BlockSpec((1, _C), lambda b, i: (0, 0)),
        ],
        out_specs=pl.BlockSpec((1, _BS3, _C), lambda b, i: (b, i, 0)),
        out_shape=jax.ShapeDtypeStruct((B, N1, _C), jnp.float32),
    )(g1, g1x, a1rows, p1T, m1, c0, w1, b1, v1, c1, v2, c2)


def _phase5_body(g_ref, gx_ref, p1_ref, v0_ref, c0_ref, v1_ref, c1_ref,
                 v2_ref, c2_ref, out_ref):
    bs = p1_ref.shape[1]
    r = bs * _K
    nfj = g_ref[0].reshape(r, _C)
    xyzj = gx_ref[0].reshape(r, 4)[:, :3]
    p1 = p1_ref[0]
    p1r = jnp.broadcast_to(p1[None, :, :], (_K, bs, 3)).reshape(r, 3)
    d = xyzj - p1r
    h = jnp.maximum(_dot(d, v0_ref[...]) + c0_ref[...], 0.0)
    h = jnp.maximum(_dot(h, v1_ref[...]) + c1_ref[...], 0.0)
    w = jnp.maximum(_dot(h, v2_ref[...]) + c2_ref[...], 0.0)
    out_ref[0] = jnp.sum((w * nfj).reshape(_K, bs, _C), axis=0)


def _phase5(g2, g2x, p1T, v0, c0, v1, c1, v2, c2):
    B, N1, _ = p1T.shape
    return pl.pallas_call(
        _phase5_body,
        grid=(B, N1 // _BS3),
        in_specs=[
            pl.BlockSpec((1, _K, _BS3, _C), lambda b, i: (b, 0, i, 0)),
            pl.BlockSpec((1, _K, _BS3, 4), lambda b, i: (b, 0, i, 0)),
            pl.BlockSpec((1, _BS3, 3), lambda b, i: (b, i, 0)),
            pl.BlockSpec((8, 3), lambda b, i: (0, 0)),
            pl.BlockSpec((1, 8), lambda b, i: (0, 0)),
            pl.BlockSpec((8, 8), lambda b, i: (0, 0)),
            pl.BlockSpec((1, 8), lambda b, i: (0, 0)),
            pl.BlockSpec((_C, 8), lambda b, i: (0, 0)),
            pl.BlockSpec((1, _C), lambda b, i: (0, 0)),
        ],
        out_specs=pl.BlockSpec((1, _BS3, _C), lambda b, i: (b, i, 0)),
        out_shape=jax.ShapeDtypeStruct((B, N1, _C), jnp.float32),
    )(g2, g2x, p1T, v0, c0, v1, c1, v2, c2)


def kernel(pc1, pc2, feature1, feature2, mlp_W0, mlp_b0, mlp_W1, mlp_b1,
           wn1_W0, wn1_b0, wn1_W1, wn1_b1, wn1_W2, wn1_b2,
           wn2_W0, wn2_b0, wn2_W1, wn2_b1, wn2_W2, wn2_b2):
    B, _, N1 = pc1.shape
    N2 = pc2.shape[2]
    p1T = jnp.transpose(pc1, (0, 2, 1))
    w0a = mlp_W0[:, :_D]
    w0b = mlp_W0[:, _D:2 * _D]
    m1 = jnp.concatenate([wn1_W0, mlp_W0[:, 2 * _D:]], axis=0)

    p2T = jnp.transpose(pc2, (0, 2, 1))
    a1rows, t2, q1aug, k1aug, k2aug = _precompute(
        feature1, feature2, p1T, p2T, w0a, w0b, mlp_b0[None])
    idx1 = _knn(q1aug, k2aug)
    idx2 = _knn(q1aug, k1aug)
    total = B * N1 * _K

    p2rows = jnp.transpose(pc2, (1, 0, 2)).reshape(3, B * N2)
    p1rows = jnp.transpose(pc1, (1, 0, 2)).reshape(3, B * N1)

    g1, g1xc = _sc_gather(t2.reshape(B * N2, _C),
                          p2rows[0], p2rows[1], p2rows[2], idx1.reshape(-1))
    g1x = jnp.transpose(g1xc, (0, 2, 1)).reshape(total, 4)
    nf = _phase3(g1.reshape(B, _K, N1, _C), g1x.reshape(B, _K, N1, 4),
                 a1rows, p1T, m1,
                 wn1_b0[None], mlp_W1, mlp_b1[None],
                 wn1_W1, wn1_b1[None], wn1_W2, wn1_b2[None])

    g2, g2xc = _sc_gather(nf.reshape(B * N1, _C),
                          p1rows[0], p1rows[1], p1rows[2], idx2.reshape(-1))
    g2x = jnp.transpose(g2xc, (0, 2, 1)).reshape(total, 4)
    out_rows = _phase5(g2.reshape(B, _K, N1, _C), g2x.reshape(B, _K, N1, 4),
                       p1T,
                       wn2_W0, wn2_b0[None], wn2_W1, wn2_b1[None],
                       wn2_W2, wn2_b2[None])
    return jnp.transpose(out_rows, (0, 2, 1))


# speedup vs baseline: 1.3920x; 1.0004x over previous
import functools

import jax
import jax.numpy as jnp
from jax import lax
from jax.experimental import pallas as pl
from jax.experimental.pallas import tpu as pltpu
from jax.experimental.pallas import tpu_sc as plsc

_K = 16
_C = 128
_D = 64
_NB0 = 512
_BSQ = 128
_BS3 = 128


def _dot(x, w):
    return lax.dot_general(x, w, (((1,), (1,)), ((), ())),
                           preferred_element_type=jnp.float32)


def _precompute_body(f1_ref, f2_ref, p1_ref, p2_ref, w0a_ref, w0b_ref,
                     b0_ref, a1_ref, t2_ref, q1_ref, k1_ref, k2_ref):
    f1 = f1_ref[0]
    f2 = f2_ref[0]
    a1 = lax.dot_general(f1, w0a_ref[...], (((0,), (1,)), ((), ())),
                         preferred_element_type=jnp.float32)
    a2 = lax.dot_general(f2, w0b_ref[...], (((0,), (1,)), ((), ())),
                         preferred_element_type=jnp.float32)
    a1_ref[0] = a1 + b0_ref[...]
    t2_ref[0] = a2
    p1 = p1_ref[0]
    p2 = p2_ref[0]
    nb = p1.shape[0]
    one = jnp.ones((nb, 1), jnp.float32)
    zero = jnp.zeros((nb, 3), jnp.float32)
    p1sq = jnp.sum(p1 * p1, axis=1, keepdims=True)
    p2sq = jnp.sum(p2 * p2, axis=1, keepdims=True)
    q1_ref[0] = jnp.concatenate([-2.0 * p1, one, p1sq, zero], axis=1)
    k1_ref[0] = jnp.concatenate([p1, p1sq, one, zero], axis=1)
    k2_ref[0] = jnp.concatenate([p2, p2sq, one, zero], axis=1)


def _precompute(feature1, feature2, p1T, p2T, w0a, w0b, b0row):
    B, D, N = feature1.shape
    grid = (B, N // _NB0)
    aug = jax.ShapeDtypeStruct((B, N, 8), jnp.float32)
    return pl.pallas_call(
        _precompute_body,
        grid=grid,
        in_specs=[
            pl.BlockSpec((1, D, _NB0), lambda b, i: (b, 0, i)),
            pl.BlockSpec((1, D, _NB0), lambda b, i: (b, 0, i)),
            pl.BlockSpec((1, _NB0, 3), lambda b, i: (b, i, 0)),
            pl.BlockSpec((1, _NB0, 3), lambda b, i: (b, i, 0)),
            pl.BlockSpec((_C, D), lambda b, i: (0, 0)),
            pl.BlockSpec((_C, D), lambda b, i: (0, 0)),
            pl.BlockSpec((1, _C), lambda b, i: (0, 0)),
        ],
        out_specs=[
            pl.BlockSpec((1, _NB0, _C), lambda b, i: (b, i, 0)),
            pl.BlockSpec((1, _NB0, _C), lambda b, i: (b, i, 0)),
            pl.BlockSpec((1, _NB0, 8), lambda b, i: (b, i, 0)),
            pl.BlockSpec((1, _NB0, 8), lambda b, i: (b, i, 0)),
            pl.BlockSpec((1, _NB0, 8), lambda b, i: (b, i, 0)),
        ],
        out_shape=[
            jax.ShapeDtypeStruct((B, N, _C), jnp.float32),
            jax.ShapeDtypeStruct((B, N, _C), jnp.float32),
            aug, aug, aug,
        ],
    )(feature1, feature2, p1T, p2T, w0a, w0b, b0row)


_S = 4


def _ce(a, b):
    return jnp.minimum(a, b), jnp.maximum(a, b)


def _sort4(a, b, c, d):
    a, b = _ce(a, b)
    c, d = _ce(c, d)
    a, c = _ce(a, c)
    b, d = _ce(b, d)
    b, c = _ce(b, c)
    return [a, b, c, d]


def _merge4(x, y):
    c0 = jnp.minimum(x[0], y[3])
    c1 = jnp.minimum(x[1], y[2])
    c2 = jnp.minimum(x[2], y[1])
    c3 = jnp.minimum(x[3], y[0])
    c0, c2 = _ce(c0, c2)
    c1, c3 = _ce(c1, c3)
    c0, c1 = _ce(c0, c1)
    c2, c3 = _ce(c2, c3)
    return [c0, c1, c2, c3]


def _knn_body(q_ref, k_ref, idx_ref, *, n_keys):
    qa = q_ref[0]
    ka = k_ref[0]
    bs = qa.shape[0]
    ng = n_keys // 128
    kq = lax.dot_general(ka[:, :3], qa[:, :3], (((1,), (1,)), ((), ())),
                         preferred_element_type=jnp.float32)
    e4 = (lax.broadcasted_iota(jnp.int32, (1, 8), 1) == 4).astype(jnp.float32)
    qn = lax.dot_general(e4, qa, (((1,), (1,)), ((), ())),
                         preferred_element_type=jnp.float32)
    kn = ka[:, 3:4]
    distT = kq + kn + qn
    keys3 = (lax.bitcast_convert_type(distT, jnp.int32).reshape(ng, 128, bs)
             & jnp.int32(-ng)) | lax.broadcasted_iota(jnp.int32,
                                                      (ng, 128, bs), 0)
    maxi = jnp.int32(2 ** 31 - 1)
    big = jnp.int32(2 ** 30)
    quads = [_sort4(keys3[4 * i], keys3[4 * i + 1],
                    keys3[4 * i + 2], keys3[4 * i + 3])
             for i in range(ng // 4)]
    while len(quads) > 1:
        quads = [_merge4(quads[2 * i], quads[2 * i + 1])
                 for i in range(len(quads) // 2)]
    stack = quads[0]
    s_iota = lax.broadcasted_iota(jnp.int32, (128, bs), 0)
    colcur = stack[0]
    cnt = jnp.zeros((128, bs), jnp.int32)
    rows = []
    for _ in range(_K):
        m = jnp.min(colcur, axis=0, keepdims=True)
        sel = colcur == m
        sstar = jnp.min(jnp.where(sel, s_iota, big),
                        axis=0, keepdims=True)
        cstar = m & jnp.int32(ng - 1)
        rows.append(cstar * 128 + sstar)
        hit = s_iota == sstar
        cnt = cnt + jnp.where(hit, 1, 0)
        refill = jnp.full((128, bs), maxi, jnp.int32)
        for s in range(1, _S):
            refill = jnp.where(cnt == s, stack[s], refill)
        colcur = jnp.where(hit, refill, colcur)
    idx = jnp.concatenate(rows, axis=0)
    idx_ref[0] = idx + pl.program_id(0) * n_keys


def _knn(q_aug, k_aug):
    B, N1, _ = q_aug.shape
    N2 = k_aug.shape[1]
    return pl.pallas_call(
        functools.partial(_knn_body, n_keys=N2),
        grid=(B, N1 // _BSQ),
        in_specs=[
            pl.BlockSpec((1, _BSQ, 8), lambda b, i: (b, i, 0)),
            pl.BlockSpec((1, N2, 8), lambda b, i: (b, 0, 0)),
        ],
        out_specs=pl.BlockSpec((1, _K, _BSQ), lambda b, i: (b, 0, i)),
        out_shape=jax.ShapeDtypeStruct((B, _K, N1), jnp.int32),
    )(q_aug, k_aug)


def _sc_gather(table, xrow, yrow, zrow, idx):
    total = idx.shape[0]
    n_chunks = total // 128
    idx2d = idx.reshape(n_chunks, 128)
    per_w = n_chunks // 32
    npts = xrow.shape[0]
    mesh = plsc.VectorSubcoreMesh(core_axis_name="c", subcore_axis_name="s")

    @functools.partial(
        pl.kernel, mesh=mesh,
        compiler_params=pltpu.CompilerParams(needs_layout_passes=False),
        out_type=[
            jax.ShapeDtypeStruct((total, _C), jnp.float32),
            jax.ShapeDtypeStruct((n_chunks, 4, 128), jnp.float32),
        ],
        scratch_types=[
            pltpu.VMEM((128,), jnp.int32),
            pltpu.VMEM((128, _C), jnp.float32),
            pltpu.VMEM((4, 128), jnp.float32),
            pltpu.VMEM((npts,), jnp.float32),
            pltpu.VMEM((npts,), jnp.float32),
            pltpu.VMEM((npts,), jnp.float32),
            pltpu.SemaphoreType.DMA,
        ],
    )
    def gk(table_hbm, x_hbm, y_hbm, z_hbm, idx_hbm, out_hbm, xyz_hbm,
           idxv, rows, xyzbuf, xv, yv, zv, sem):
        wid = lax.axis_index("s") * 2 + lax.axis_index("c")
        pltpu.sync_copy(x_hbm, xv)
        pltpu.sync_copy(y_hbm, yv)
        pltpu.sync_copy(z_hbm, zv)

        def body(c, carry):
            row = wid * per_w + c
            pltpu.sync_copy(idx_hbm.at[row], idxv)
            pltpu.async_copy(table_hbm.at[idxv], rows, sem).wait()
            for g in range(8):
                iv = idxv[pl.ds(g * 16, 16)]
                xyzbuf[0, pl.ds(g * 16, 16)] = plsc.load_gather(xv, [iv])
                xyzbuf[1, pl.ds(g * 16, 16)] = plsc.load_gather(yv, [iv])
                xyzbuf[2, pl.ds(g * 16, 16)] = plsc.load_gather(zv, [iv])
            pltpu.sync_copy(rows, out_hbm.at[pl.ds(row * 128, 128)])
            pltpu.sync_copy(xyzbuf, xyz_hbm.at[row])
            return carry

        lax.fori_loop(0, per_w, body, 0)

    return gk(table, xrow, yrow, zrow, idx2d)


def _phase3_body(g_ref, gx_ref, a1_ref, p1_ref, m1_ref, c0_ref, w1_ref,
                 b1_ref, v1_ref, c1_ref, v2_ref, c2_ref, out_ref):
    bs = p1_ref.shape[1]
    r = bs * _K
    a2 = g_ref[0].reshape(r, _C)
    xyzj = gx_ref[0].reshape(r, 4)[:, :3]
    p1 = p1_ref[0]
    p1r = jnp.broadcast_to(p1[None, :, :], (_K, bs, 3)).reshape(r, 3)
    d = xyzj - p1r
    t = _dot(d, m1_ref[...])
    h = jnp.maximum(t[:, :8] + c0_ref[...], 0.0)
    dirproj = t[:, 8:8 + _C]
    a1 = a1_ref[0]
    a1r = jnp.broadcast_to(a1[None, :, :], (_K, bs, _C)).reshape(r, _C)
    x = a1r + a2 + dirproj
    x = jnp.where(x >= 0.0, x, 0.1 * x)
    y = _dot(x, w1_ref[...]) + b1_ref[...]
    y = jnp.where(y >= 0.0, y, 0.1 * y)
    h = jnp.maximum(_dot(h, v1_ref[...]) + c1_ref[...], 0.0)
    w = jnp.maximum(_dot(h, v2_ref[...]) + c2_ref[...], 0.0)
    out_ref[0] = jnp.sum((w * y).reshape(_K, bs, _C), axis=0)


def _phase3(g1, g1x, a1rows, p1T, m1, c0, w1, b1, v1, c1, v2, c2):
    B, N1, _ = p1T.shape
    return pl.pallas_call(
        _phase3_body,
        grid=(B, N1 // _BS3),
        in_specs=[
            pl.BlockSpec((1, _K, _BS3, _C), lambda b, i: (b, 0, i, 0)),
            pl.BlockSpec((1, _K, _BS3, 4), lambda b, i: (b, 0, i, 0)),
            pl.BlockSpec((1, _BS3, _C), lambda b, i: (b, i, 0)),
            pl.BlockSpec((1, _BS3, 3), lambda b, i: (b, i, 0)),
            pl.BlockSpec((8 + _C, 3), lambda b, i: (0, 0)),
            pl.BlockSpec((1, 8), lambda b, i: (0, 0)),
            pl.BlockSpec((_C, _C), lambda b, i: (0, 0)),
            pl.BlockSpec((1, _C), lambda b, i: (0, 0)),
            pl.BlockSpec((8, 8), lambda b, i: (0, 0)),
            pl.BlockSpec((1, 8), lambda b, i: (0, 0)),
            pl.BlockSpec((_C, 8), lambda b, i: (0, 0)),
            pl.BlockSpec((1, _C), lambda b, i: (0, 0)),
        ],
        out_specs=pl.BlockSpec((1, _BS3, _C), lambda b, i: (b, i, 0)),
        out_shape=jax.ShapeDtypeStruct((B, N1, _C), jnp.float32),
    )(g1, g1x, a1rows, p1T, m1, c0, w1, b1, v1, c1, v2, c2)


def _phase5_body(g_ref, gx_ref, p1_ref, v0_ref, c0_ref, v1_ref, c1_ref,
                 v2_ref, c2_ref, out_ref):
    bs = p1_ref.shape[1]
    r = bs * _K
    nfj = g_ref[0].reshape(r, _C)
    xyzj = gx_ref[0].reshape(r, 4)[:, :3]
    p1 = p1_ref[0]
    p1r = jnp.broadcast_to(p1[None, :, :], (_K, bs, 3)).reshape(r, 3)
    d = xyzj - p1r
    h = jnp.maximum(_dot(d, v0_ref[...]) + c0_ref[...], 0.0)
    h = jnp.maximum(_dot(h, v1_ref[...]) + c1_ref[...], 0.0)
    w = jnp.maximum(_dot(h, v2_ref[...]) + c2_ref[...], 0.0)
    out_ref[0] = jnp.sum((w * nfj).reshape(_K, bs, _C), axis=0)


def _phase5(g2, g2x, p1T, v0, c0, v1, c1, v2, c2):
    B, N1, _ = p1T.shape
    return pl.pallas_call(
        _phase5_body,
        grid=(B, N1 // _BS3),
        in_specs=[
            pl.BlockSpec((1, _K, _BS3, _C), lambda b, i: (b, 0, i, 0)),
            pl.BlockSpec((1, _K, _BS3, 4), lambda b, i: (b, 0, i, 0)),
            pl.BlockSpec((1, _BS3, 3), lambda b, i: (b, i, 0)),
            pl.BlockSpec((8, 3), lambda b, i: (0, 0)),
            pl.BlockSpec((1, 8), lambda b, i: (0, 0)),
            pl.BlockSpec((8, 8), lambda b, i: (0, 0)),
            pl.BlockSpec((1, 8), lambda b, i: (0, 0)),
            pl.BlockSpec((_C, 8), lambda b, i: (0, 0)),
            pl.BlockSpec((1, _C), lambda b, i: (0, 0)),
        ],
        out_specs=pl.BlockSpec((1, _BS3, _C), lambda b, i: (b, i, 0)),
        out_shape=jax.ShapeDtypeStruct((B, N1, _C), jnp.float32),
    )(g2, g2x, p1T, v0, c0, v1, c1, v2, c2)


def kernel(pc1, pc2, feature1, feature2, mlp_W0, mlp_b0, mlp_W1, mlp_b1,
           wn1_W0, wn1_b0, wn1_W1, wn1_b1, wn1_W2, wn1_b2,
           wn2_W0, wn2_b0, wn2_W1, wn2_b1, wn2_W2, wn2_b2):
    B, _, N1 = pc1.shape
    N2 = pc2.shape[2]
    p1T = jnp.transpose(pc1, (0, 2, 1))
    w0a = mlp_W0[:, :_D]
    w0b = mlp_W0[:, _D:2 * _D]
    m1 = jnp.concatenate([wn1_W0, mlp_W0[:, 2 * _D:]], axis=0)

    p2T = jnp.transpose(pc2, (0, 2, 1))
    a1rows, t2, q1aug, k1aug, k2aug = _precompute(
        feature1, feature2, p1T, p2T, w0a, w0b, mlp_b0[None])
    total = B * N1 * _K
    p2rows = jnp.transpose(pc2, (1, 0, 2)).reshape(3, B * N2)
    p1rows = jnp.transpose(pc1, (1, 0, 2)).reshape(3, B * N1)

    idx1 = _knn(q1aug, k2aug)
    g1, g1xc = _sc_gather(t2.reshape(B * N2, _C),
                          p2rows[0], p2rows[1], p2rows[2], idx1.reshape(-1))
    idx2 = _knn(q1aug, k1aug)
    g1x = jnp.transpose(g1xc, (0, 2, 1)).reshape(total, 4)
    nf = _phase3(g1.reshape(B, _K, N1, _C), g1x.reshape(B, _K, N1, 4),
                 a1rows, p1T, m1,
                 wn1_b0[None], mlp_W1, mlp_b1[None],
                 wn1_W1, wn1_b1[None], wn1_W2, wn1_b2[None])

    g2, g2xc = _sc_gather(nf.reshape(B * N1, _C),
                          p1rows[0], p1rows[1], p1rows[2], idx2.reshape(-1))
    g2x = jnp.transpose(g2xc, (0, 2, 1)).reshape(total, 4)
    out_rows = _phase5(g2.reshape(B, _K, N1, _C), g2x.reshape(B, _K, N1, 4),
                       p1T,
                       wn2_W0, wn2_b0[None], wn2_W1, wn2_b1[None],
                       wn2_W2, wn2_b2[None])
    return jnp.transpose(out_rows, (0, 2, 1))


# speedup vs baseline: 1.4738x; 1.0588x over previous
import functools

import jax
import jax.numpy as jnp
from jax import lax
from jax.experimental import pallas as pl
from jax.experimental.pallas import tpu as pltpu
from jax.experimental.pallas import tpu_sc as plsc

_K = 16
_C = 128
_D = 64
_NB0 = 512
_BSQ = 128
_BS3 = 128


def _dot(x, w):
    return lax.dot_general(x, w, (((1,), (1,)), ((), ())),
                           preferred_element_type=jnp.float32)


def _precompute_body(f1_ref, f2_ref, p1_ref, p2_ref, w0a_ref, w0b_ref,
                     b0_ref, a1_ref, t2_ref, q1_ref, k1_ref, k2_ref):
    f1 = f1_ref[0]
    f2 = f2_ref[0]
    a1 = lax.dot_general(f1, w0a_ref[...], (((0,), (1,)), ((), ())),
                         preferred_element_type=jnp.float32)
    a2 = lax.dot_general(f2, w0b_ref[...], (((0,), (1,)), ((), ())),
                         preferred_element_type=jnp.float32)
    a1_ref[0] = a1 + b0_ref[...]
    t2_ref[0] = a2
    p1 = p1_ref[0]
    p2 = p2_ref[0]
    nb = p1.shape[0]
    one = jnp.ones((nb, 1), jnp.float32)
    zero = jnp.zeros((nb, 3), jnp.float32)
    p1sq = jnp.sum(p1 * p1, axis=1, keepdims=True)
    p2sq = jnp.sum(p2 * p2, axis=1, keepdims=True)
    q1_ref[0] = jnp.concatenate([-2.0 * p1, one, p1sq, zero], axis=1)
    k1_ref[0] = jnp.concatenate([p1, p1sq, one, zero], axis=1)
    k2_ref[0] = jnp.concatenate([p2, p2sq, one, zero], axis=1)


def _precompute(feature1, feature2, p1T, p2T, w0a, w0b, b0row):
    B, D, N = feature1.shape
    grid = (B, N // _NB0)
    aug = jax.ShapeDtypeStruct((B, N, 8), jnp.float32)
    return pl.pallas_call(
        _precompute_body,
        grid=grid,
        in_specs=[
            pl.BlockSpec((1, D, _NB0), lambda b, i: (b, 0, i)),
            pl.BlockSpec((1, D, _NB0), lambda b, i: (b, 0, i)),
            pl.BlockSpec((1, _NB0, 3), lambda b, i: (b, i, 0)),
            pl.BlockSpec((1, _NB0, 3), lambda b, i: (b, i, 0)),
            pl.BlockSpec((_C, D), lambda b, i: (0, 0)),
            pl.BlockSpec((_C, D), lambda b, i: (0, 0)),
            pl.BlockSpec((1, _C), lambda b, i: (0, 0)),
        ],
        out_specs=[
            pl.BlockSpec((1, _NB0, _C), lambda b, i: (b, i, 0)),
            pl.BlockSpec((1, _NB0, _C), lambda b, i: (b, i, 0)),
            pl.BlockSpec((1, _NB0, 8), lambda b, i: (b, i, 0)),
            pl.BlockSpec((1, _NB0, 8), lambda b, i: (b, i, 0)),
            pl.BlockSpec((1, _NB0, 8), lambda b, i: (b, i, 0)),
        ],
        out_shape=[
            jax.ShapeDtypeStruct((B, N, _C), jnp.float32),
            jax.ShapeDtypeStruct((B, N, _C), jnp.float32),
            aug, aug, aug,
        ],
    )(feature1, feature2, p1T, p2T, w0a, w0b, b0row)


_S = 4


def _ce(a, b):
    return jnp.minimum(a, b), jnp.maximum(a, b)


def _sort4(a, b, c, d):
    a, b = _ce(a, b)
    c, d = _ce(c, d)
    a, c = _ce(a, c)
    b, d = _ce(b, d)
    b, c = _ce(b, c)
    return [a, b, c, d]


def _merge4(x, y):
    c0 = jnp.minimum(x[0], y[3])
    c1 = jnp.minimum(x[1], y[2])
    c2 = jnp.minimum(x[2], y[1])
    c3 = jnp.minimum(x[3], y[0])
    c0, c2 = _ce(c0, c2)
    c1, c3 = _ce(c1, c3)
    c0, c1 = _ce(c0, c1)
    c2, c3 = _ce(c2, c3)
    return [c0, c1, c2, c3]


def _knn_body(q_ref, k_ref, idx_ref, *, n_keys):
    qa = q_ref[0]
    ka = k_ref[0]
    bs = qa.shape[0]
    ng = n_keys // 128
    kq = lax.dot_general(ka[:, :3], qa[:, :3], (((1,), (1,)), ((), ())),
                         preferred_element_type=jnp.float32)
    e4 = (lax.broadcasted_iota(jnp.int32, (1, 8), 1) == 4).astype(jnp.float32)
    qn = lax.dot_general(e4, qa, (((1,), (1,)), ((), ())),
                         preferred_element_type=jnp.float32)
    kn = ka[:, 3:4]
    distT = kq + kn + qn
    keys3 = (lax.bitcast_convert_type(distT, jnp.int32).reshape(ng, 128, bs)
             & jnp.int32(-ng)) | lax.broadcasted_iota(jnp.int32,
                                                      (ng, 128, bs), 0)
    maxi = jnp.int32(2 ** 31 - 1)
    big = jnp.int32(2 ** 30)
    quads = [_sort4(keys3[4 * i], keys3[4 * i + 1],
                    keys3[4 * i + 2], keys3[4 * i + 3])
             for i in range(ng // 4)]
    while len(quads) > 1:
        quads = [_merge4(quads[2 * i], quads[2 * i + 1])
                 for i in range(len(quads) // 2)]
    stack = quads[0]
    s_iota = lax.broadcasted_iota(jnp.int32, (128, bs), 0)
    colcur = stack[0]
    cnt = jnp.zeros((128, bs), jnp.int32)
    rows = []
    for _ in range(_K):
        m = jnp.min(colcur, axis=0, keepdims=True)
        sel = colcur == m
        sstar = jnp.min(jnp.where(sel, s_iota, big),
                        axis=0, keepdims=True)
        cstar = m & jnp.int32(ng - 1)
        rows.append(cstar * 128 + sstar)
        hit = s_iota == sstar
        cnt = cnt + jnp.where(hit, 1, 0)
        refill = jnp.full((128, bs), maxi, jnp.int32)
        for s in range(1, _S):
            refill = jnp.where(cnt == s, stack[s], refill)
        colcur = jnp.where(hit, refill, colcur)
    idx = jnp.concatenate(rows, axis=0)
    idx_ref[0] = idx + pl.program_id(0) * n_keys


def _knn(q_aug, k_aug):
    B, N1, _ = q_aug.shape
    N2 = k_aug.shape[1]
    return pl.pallas_call(
        functools.partial(_knn_body, n_keys=N2),
        grid=(B, N1 // _BSQ),
        in_specs=[
            pl.BlockSpec((1, _BSQ, 8), lambda b, i: (b, i, 0)),
            pl.BlockSpec((1, N2, 8), lambda b, i: (b, 0, 0)),
        ],
        out_specs=pl.BlockSpec((1, _K, _BSQ), lambda b, i: (b, 0, i)),
        out_shape=jax.ShapeDtypeStruct((B, _K, N1), jnp.int32),
    )(q_aug, k_aug)


def _sc_gather(table, xrow, yrow, zrow, idx):
    total = idx.shape[0]
    n_chunks = total // 128
    idx2d = idx.reshape(n_chunks, 128)
    per_w = n_chunks // 32
    npts = xrow.shape[0]
    mesh = plsc.VectorSubcoreMesh(core_axis_name="c", subcore_axis_name="s")

    @functools.partial(
        pl.kernel, mesh=mesh,
        compiler_params=pltpu.CompilerParams(needs_layout_passes=False),
        out_type=[
            jax.ShapeDtypeStruct((total, _C), jnp.float32),
            jax.ShapeDtypeStruct((n_chunks, 4, 128), jnp.float32),
        ],
        scratch_types=[
            pltpu.VMEM((128,), jnp.int32),
            pltpu.VMEM((128,), jnp.int32),
            pltpu.VMEM((128, _C), jnp.float32),
            pltpu.VMEM((128, _C), jnp.float32),
            pltpu.VMEM((4, 128), jnp.float32),
            pltpu.VMEM((4, 128), jnp.float32),
            pltpu.VMEM((npts,), jnp.float32),
            pltpu.VMEM((npts,), jnp.float32),
            pltpu.VMEM((npts,), jnp.float32),
            pltpu.SemaphoreType.DMA,
            pltpu.SemaphoreType.DMA,
        ],
    )
    def gk(table_hbm, x_hbm, y_hbm, z_hbm, idx_hbm, out_hbm, xyz_hbm,
           idxv0, idxv1, rows0, rows1, xyz0, xyz1, xv, yv, zv, sem0, sem1):
        wid = lax.axis_index("s") * 2 + lax.axis_index("c")
        base = wid * per_w
        pltpu.sync_copy(x_hbm, xv)
        pltpu.sync_copy(y_hbm, yv)
        pltpu.sync_copy(z_hbm, zv)

        def start(row, idxv, rows, sem):
            pltpu.sync_copy(idx_hbm.at[row], idxv)
            pltpu.async_copy(table_hbm.at[idxv], rows, sem)

        def finish(row, idxv, rows, xyzbuf, sem):
            pltpu.make_async_copy(table_hbm.at[idxv], rows, sem).wait()
            for g in range(8):
                iv = idxv[pl.ds(g * 16, 16)]
                xyzbuf[0, pl.ds(g * 16, 16)] = plsc.load_gather(xv, [iv])
                xyzbuf[1, pl.ds(g * 16, 16)] = plsc.load_gather(yv, [iv])
                xyzbuf[2, pl.ds(g * 16, 16)] = plsc.load_gather(zv, [iv])
            pltpu.sync_copy(rows, out_hbm.at[pl.ds(row * 128, 128)])
            pltpu.sync_copy(xyzbuf, xyz_hbm.at[row])

        start(base, idxv0, rows0, sem0)
        start(base + 1, idxv1, rows1, sem1)

        def body(i, carry):
            row = base + 2 * i
            finish(row, idxv0, rows0, xyz0, sem0)
            start(row + 2, idxv0, rows0, sem0)
            finish(row + 1, idxv1, rows1, xyz1, sem1)
            start(row + 3, idxv1, rows1, sem1)
            return carry

        lax.fori_loop(0, per_w // 2 - 1, body, 0)
        last = base + per_w - 2
        finish(last, idxv0, rows0, xyz0, sem0)
        finish(last + 1, idxv1, rows1, xyz1, sem1)

    return gk(table, xrow, yrow, zrow, idx2d)


def _phase3_body(g_ref, gx_ref, a1_ref, p1_ref, m1_ref, c0_ref, w1_ref,
                 b1_ref, v1_ref, c1_ref, v2_ref, c2_ref, out_ref):
    bs = p1_ref.shape[1]
    r = bs * _K
    a2 = g_ref[0].reshape(r, _C)
    xyzj = gx_ref[0].reshape(r, 4)[:, :3]
    p1 = p1_ref[0]
    p1r = jnp.broadcast_to(p1[None, :, :], (_K, bs, 3)).reshape(r, 3)
    d = xyzj - p1r
    t = _dot(d, m1_ref[...])
    h = jnp.maximum(t[:, :8] + c0_ref[...], 0.0)
    dirproj = t[:, 8:8 + _C]
    a1 = a1_ref[0]
    a1r = jnp.broadcast_to(a1[None, :, :], (_K, bs, _C)).reshape(r, _C)
    x = a1r + a2 + dirproj
    x = jnp.where(x >= 0.0, x, 0.1 * x)
    y = _dot(x, w1_ref[...]) + b1_ref[...]
    y = jnp.where(y >= 0.0, y, 0.1 * y)
    h = jnp.maximum(_dot(h, v1_ref[...]) + c1_ref[...], 0.0)
    w = jnp.maximum(_dot(h, v2_ref[...]) + c2_ref[...], 0.0)
    out_ref[0] = jnp.sum((w * y).reshape(_K, bs, _C), axis=0)


def _phase3(g1, g1x, a1rows, p1T, m1, c0, w1, b1, v1, c1, v2, c2):
    B, N1, _ = p1T.shape
    return pl.pallas_call(
        _phase3_body,
        grid=(B, N1 // _BS3),
        in_specs=[
            pl.BlockSpec((1, _K, _BS3, _C), lambda b, i: (b, 0, i, 0)),
            pl.BlockSpec((1, _K, _BS3, 4), lambda b, i: (b, 0, i, 0)),
            pl.BlockSpec((1, _BS3, _C), lambda b, i: (b, i, 0)),
            pl.BlockSpec((1, _BS3, 3), lambda b, i: (b, i, 0)),
            pl.BlockSpec((8 + _C, 3), lambda b, i: (0, 0)),
            pl.BlockSpec((1, 8), lambda b, i: (0, 0)),
            pl.BlockSpec((_C, _C), lambda b, i: (0, 0)),
            pl.BlockSpec((1, _C), lambda b, i: (0, 0)),
            pl.BlockSpec((8, 8), lambda b, i: (0, 0)),
            pl.BlockSpec((1, 8), lambda b, i: (0, 0)),
            pl.BlockSpec((_C, 8), lambda b, i: (0, 0)),
            pl.BlockSpec((1, _C), lambda b, i: (0, 0)),
        ],
        out_specs=pl.BlockSpec((1, _BS3, _C), lambda b, i: (b, i, 0)),
        out_shape=jax.ShapeDtypeStruct((B, N1, _C), jnp.float32),
    )(g1, g1x, a1rows, p1T, m1, c0, w1, b1, v1, c1, v2, c2)


def _phase5_body(g_ref, gx_ref, p1_ref, v0_ref, c0_ref, v1_ref, c1_ref,
                 v2_ref, c2_ref, out_ref):
    bs = p1_ref.shape[1]
    r = bs * _K
    nfj = g_ref[0].reshape(r, _C)
    xyzj = gx_ref[0].reshape(r, 4)[:, :3]
    p1 = p1_ref[0]
    p1r = jnp.broadcast_to(p1[None, :, :], (_K, bs, 3)).reshape(r, 3)
    d = xyzj - p1r
    h = jnp.maximum(_dot(d, v0_ref[...]) + c0_ref[...], 0.0)
    h = jnp.maximum(_dot(h, v1_ref[...]) + c1_ref[...], 0.0)
    w = jnp.maximum(_dot(h, v2_ref[...]) + c2_ref[...], 0.0)
    out_ref[0] = jnp.sum((w * nfj).reshape(_K, bs, _C), axis=0)


def _phase5(g2, g2x, p1T, v0, c0, v1, c1, v2, c2):
    B, N1, _ = p1T.shape
    return pl.pallas_call(
        _phase5_body,
        grid=(B, N1 // _BS3),
        in_specs=[
            pl.BlockSpec((1, _K, _BS3, _C), lambda b, i: (b, 0, i, 0)),
            pl.BlockSpec((1, _K, _BS3, 4), lambda b, i: (b, 0, i, 0)),
            pl.BlockSpec((1, _BS3, 3), lambda b, i: (b, i, 0)),
            pl.BlockSpec((8, 3), lambda b, i: (0, 0)),
            pl.BlockSpec((1, 8), lambda b, i: (0, 0)),
            pl.BlockSpec((8, 8), lambda b, i: (0, 0)),
            pl.BlockSpec((1, 8), lambda b, i: (0, 0)),
            pl.BlockSpec((_C, 8), lambda b, i: (0, 0)),
            pl.BlockSpec((1, _C), lambda b, i: (0, 0)),
        ],
        out_specs=pl.BlockSpec((1, _BS3, _C), lambda b, i: (b, i, 0)),
        out_shape=jax.ShapeDtypeStruct((B, N1, _C), jnp.float32),
    )(g2, g2x, p1T, v0, c0, v1, c1, v2, c2)


def kernel(pc1, pc2, feature1, feature2, mlp_W0, mlp_b0, mlp_W1, mlp_b1,
           wn1_W0, wn1_b0, wn1_W1, wn1_b1, wn1_W2, wn1_b2,
           wn2_W0, wn2_b0, wn2_W1, wn2_b1, wn2_W2, wn2_b2):
    B, _, N1 = pc1.shape
    N2 = pc2.shape[2]
    p1T = jnp.transpose(pc1, (0, 2, 1))
    w0a = mlp_W0[:, :_D]
    w0b = mlp_W0[:, _D:2 * _D]
    m1 = jnp.concatenate([wn1_W0, mlp_W0[:, 2 * _D:]], axis=0)

    p2T = jnp.transpose(pc2, (0, 2, 1))
    a1rows, t2, q1aug, k1aug, k2aug = _precompute(
        feature1, feature2, p1T, p2T, w0a, w0b, mlp_b0[None])
    total = B * N1 * _K
    p2rows = jnp.transpose(pc2, (1, 0, 2)).reshape(3, B * N2)
    p1rows = jnp.transpose(pc1, (1, 0, 2)).reshape(3, B * N1)

    idx1 = _knn(q1aug, k2aug)
    g1, g1xc = _sc_gather(t2.reshape(B * N2, _C),
                          p2rows[0], p2rows[1], p2rows[2], idx1.reshape(-1))
    idx2 = _knn(q1aug, k1aug)
    g1x = jnp.transpose(g1xc, (0, 2, 1)).reshape(total, 4)
    nf = _phase3(g1.reshape(B, _K, N1, _C), g1x.reshape(B, _K, N1, 4),
                 a1rows, p1T, m1,
                 wn1_b0[None], mlp_W1, mlp_b1[None],
                 wn1_W1, wn1_b1[None], wn1_W2, wn1_b2[None])

    g2, g2xc = _sc_gather(nf.reshape(B * N1, _C),
                          p1rows[0], p1rows[1], p1rows[2], idx2.reshape(-1))
    g2x = jnp.transpose(g2xc, (0, 2, 1)).reshape(total, 4)
    out_rows = _phase5(g2.reshape(B, _K, N1, _C), g2x.reshape(B, _K, N1, 4),
                       p1T,
                       wn2_W0, wn2_b0[None], wn2_W1, wn2_b1[None],
                       wn2_W2, wn2_b2[None])
    return jnp.transpose(out_rows, (0, 2, 1))


# speedup vs baseline: 1.5459x; 1.0489x over previous
import functools

import jax
import jax.numpy as jnp
from jax import lax
from jax.experimental import pallas as pl
from jax.experimental.pallas import tpu as pltpu
from jax.experimental.pallas import tpu_sc as plsc

_K = 16
_C = 128
_D = 64
_NB0 = 512
_BSQ = 256
_BS3 = 128


def _dot(x, w):
    return lax.dot_general(x, w, (((1,), (1,)), ((), ())),
                           preferred_element_type=jnp.float32)


def _precompute_body(f1_ref, f2_ref, p1_ref, p2_ref, w0a_ref, w0b_ref,
                     b0_ref, a1_ref, t2_ref, q1_ref, k1_ref, k2_ref):
    f1 = f1_ref[0]
    f2 = f2_ref[0]
    a1 = lax.dot_general(f1, w0a_ref[...], (((0,), (1,)), ((), ())),
                         preferred_element_type=jnp.float32)
    a2 = lax.dot_general(f2, w0b_ref[...], (((0,), (1,)), ((), ())),
                         preferred_element_type=jnp.float32)
    a1_ref[0] = a1 + b0_ref[...]
    t2_ref[0] = a2
    p1 = p1_ref[0]
    p2 = p2_ref[0]
    nb = p1.shape[0]
    one = jnp.ones((nb, 1), jnp.float32)
    zero = jnp.zeros((nb, 3), jnp.float32)
    p1sq = jnp.sum(p1 * p1, axis=1, keepdims=True)
    p2sq = jnp.sum(p2 * p2, axis=1, keepdims=True)
    q1_ref[0] = jnp.concatenate([-2.0 * p1, one, p1sq, zero], axis=1)
    k1_ref[0] = jnp.concatenate([p1, p1sq, one, zero], axis=1)
    k2_ref[0] = jnp.concatenate([p2, p2sq, one, zero], axis=1)


def _precompute(feature1, feature2, p1T, p2T, w0a, w0b, b0row):
    B, D, N = feature1.shape
    grid = (B, N // _NB0)
    aug = jax.ShapeDtypeStruct((B, N, 8), jnp.float32)
    return pl.pallas_call(
        _precompute_body,
        grid=grid,
        in_specs=[
            pl.BlockSpec((1, D, _NB0), lambda b, i: (b, 0, i)),
            pl.BlockSpec((1, D, _NB0), lambda b, i: (b, 0, i)),
            pl.BlockSpec((1, _NB0, 3), lambda b, i: (b, i, 0)),
            pl.BlockSpec((1, _NB0, 3), lambda b, i: (b, i, 0)),
            pl.BlockSpec((_C, D), lambda b, i: (0, 0)),
            pl.BlockSpec((_C, D), lambda b, i: (0, 0)),
            pl.BlockSpec((1, _C), lambda b, i: (0, 0)),
        ],
        out_specs=[
            pl.BlockSpec((1, _NB0, _C), lambda b, i: (b, i, 0)),
            pl.BlockSpec((1, _NB0, _C), lambda b, i: (b, i, 0)),
            pl.BlockSpec((1, _NB0, 8), lambda b, i: (b, i, 0)),
            pl.BlockSpec((1, _NB0, 8), lambda b, i: (b, i, 0)),
            pl.BlockSpec((1, _NB0, 8), lambda b, i: (b, i, 0)),
        ],
        out_shape=[
            jax.ShapeDtypeStruct((B, N, _C), jnp.float32),
            jax.ShapeDtypeStruct((B, N, _C), jnp.float32),
            aug, aug, aug,
        ],
    )(feature1, feature2, p1T, p2T, w0a, w0b, b0row)


_S = 4


def _ce(a, b):
    return jnp.minimum(a, b), jnp.maximum(a, b)


def _sort4(a, b, c, d):
    a, b = _ce(a, b)
    c, d = _ce(c, d)
    a, c = _ce(a, c)
    b, d = _ce(b, d)
    b, c = _ce(b, c)
    return [a, b, c, d]


def _merge4(x, y):
    c0 = jnp.minimum(x[0], y[3])
    c1 = jnp.minimum(x[1], y[2])
    c2 = jnp.minimum(x[2], y[1])
    c3 = jnp.minimum(x[3], y[0])
    c0, c2 = _ce(c0, c2)
    c1, c3 = _ce(c1, c3)
    c0, c1 = _ce(c0, c1)
    c2, c3 = _ce(c2, c3)
    return [c0, c1, c2, c3]


def _knn_body(q_ref, k_ref, idx_ref, *, n_keys):
    qa = q_ref[0]
    ka = k_ref[0]
    bs = qa.shape[0]
    ng = n_keys // 128
    kq = lax.dot_general(ka[:, :3], qa[:, :3], (((1,), (1,)), ((), ())),
                         preferred_element_type=jnp.float32)
    e4 = (lax.broadcasted_iota(jnp.int32, (1, 8), 1) == 4).astype(jnp.float32)
    qn = lax.dot_general(e4, qa, (((1,), (1,)), ((), ())),
                         preferred_element_type=jnp.float32)
    kn = ka[:, 3:4]
    distT = kq + kn + qn
    keys3 = (lax.bitcast_convert_type(distT, jnp.int32).reshape(ng, 128, bs)
             & jnp.int32(-ng)) | lax.broadcasted_iota(jnp.int32,
                                                      (ng, 128, bs), 0)
    maxi = jnp.int32(2 ** 31 - 1)
    big = jnp.int32(2 ** 30)
    quads = [_sort4(keys3[4 * i], keys3[4 * i + 1],
                    keys3[4 * i + 2], keys3[4 * i + 3])
             for i in range(ng // 4)]
    while len(quads) > 1:
        quads = [_merge4(quads[2 * i], quads[2 * i + 1])
                 for i in range(len(quads) // 2)]
    stack = quads[0]
    s_iota = lax.broadcasted_iota(jnp.int32, (128, bs), 0)
    colcur = stack[0]
    cnt = jnp.zeros((128, bs), jnp.int32)
    rows = []
    for _ in range(_K):
        m = jnp.min(colcur, axis=0, keepdims=True)
        sel = colcur == m
        sstar = jnp.min(jnp.where(sel, s_iota, big),
                        axis=0, keepdims=True)
        cstar = m & jnp.int32(ng - 1)
        rows.append(cstar * 128 + sstar)
        hit = s_iota == sstar
        cnt = cnt + jnp.where(hit, 1, 0)
        refill = jnp.full((128, bs), maxi, jnp.int32)
        for s in range(1, _S):
            refill = jnp.where(cnt == s, stack[s], refill)
        colcur = jnp.where(hit, refill, colcur)
    idx = jnp.concatenate(rows, axis=0)
    idx_ref[0] = idx + pl.program_id(0) * n_keys


def _knn(q_aug, k_aug):
    B, N1, _ = q_aug.shape
    N2 = k_aug.shape[1]
    return pl.pallas_call(
        functools.partial(_knn_body, n_keys=N2),
        grid=(B, N1 // _BSQ),
        in_specs=[
            pl.BlockSpec((1, _BSQ, 8), lambda b, i: (b, i, 0)),
            pl.BlockSpec((1, N2, 8), lambda b, i: (b, 0, 0)),
        ],
        out_specs=pl.BlockSpec((1, _K, _BSQ), lambda b, i: (b, 0, i)),
        out_shape=jax.ShapeDtypeStruct((B, _K, N1), jnp.int32),
    )(q_aug, k_aug)


def _sc_gather(table, xrow, yrow, zrow, idx):
    total = idx.shape[0]
    n_chunks = total // 128
    idx2d = idx.reshape(n_chunks, 128)
    per_w = n_chunks // 32
    npts = xrow.shape[0]
    mesh = plsc.VectorSubcoreMesh(core_axis_name="c", subcore_axis_name="s")

    @functools.partial(
        pl.kernel, mesh=mesh,
        compiler_params=pltpu.CompilerParams(needs_layout_passes=False),
        out_type=[
            jax.ShapeDtypeStruct((total, _C), jnp.float32),
            jax.ShapeDtypeStruct((n_chunks, 4, 128), jnp.float32),
        ],
        scratch_types=[
            pltpu.VMEM((128,), jnp.int32),
            pltpu.VMEM((128,), jnp.int32),
            pltpu.VMEM((128, _C), jnp.float32),
            pltpu.VMEM((128, _C), jnp.float32),
            pltpu.VMEM((4, 128), jnp.float32),
            pltpu.VMEM((4, 128), jnp.float32),
            pltpu.VMEM((npts,), jnp.float32),
            pltpu.VMEM((npts,), jnp.float32),
            pltpu.VMEM((npts,), jnp.float32),
            pltpu.SemaphoreType.DMA,
            pltpu.SemaphoreType.DMA,
        ],
    )
    def gk(table_hbm, x_hbm, y_hbm, z_hbm, idx_hbm, out_hbm, xyz_hbm,
           idxv0, idxv1, rows0, rows1, xyz0, xyz1, xv, yv, zv, sem0, sem1):
        wid = lax.axis_index("s") * 2 + lax.axis_index("c")
        base = wid * per_w
        pltpu.sync_copy(x_hbm, xv)
        pltpu.sync_copy(y_hbm, yv)
        pltpu.sync_copy(z_hbm, zv)

        def start(row, idxv, rows, sem):
            pltpu.sync_copy(idx_hbm.at[row], idxv)
            pltpu.async_copy(table_hbm.at[idxv], rows, sem)

        def finish(row, idxv, rows, xyzbuf, sem):
            pltpu.make_async_copy(table_hbm.at[idxv], rows, sem).wait()
            for g in range(8):
                iv = idxv[pl.ds(g * 16, 16)]
                xyzbuf[0, pl.ds(g * 16, 16)] = plsc.load_gather(xv, [iv])
                xyzbuf[1, pl.ds(g * 16, 16)] = plsc.load_gather(yv, [iv])
                xyzbuf[2, pl.ds(g * 16, 16)] = plsc.load_gather(zv, [iv])
            pltpu.sync_copy(rows, out_hbm.at[pl.ds(row * 128, 128)])
            pltpu.sync_copy(xyzbuf, xyz_hbm.at[row])

        start(base, idxv0, rows0, sem0)
        start(base + 1, idxv1, rows1, sem1)

        def body(i, carry):
            row = base + 2 * i
            finish(row, idxv0, rows0, xyz0, sem0)
            start(row + 2, idxv0, rows0, sem0)
            finish(row + 1, idxv1, rows1, xyz1, sem1)
            start(row + 3, idxv1, rows1, sem1)
            return carry

        lax.fori_loop(0, per_w // 2 - 1, body, 0)
        last = base + per_w - 2
        finish(last, idxv0, rows0, xyz0, sem0)
        finish(last + 1, idxv1, rows1, xyz1, sem1)

    return gk(table, xrow, yrow, zrow, idx2d)


def _phase3_body(g_ref, gx_ref, a1_ref, p1_ref, m1_ref, c0_ref, w1_ref,
                 b1_ref, v1_ref, c1_ref, v2_ref, c2_ref, out_ref):
    bs = p1_ref.shape[1]
    r = bs * _K
    a2 = g_ref[0].reshape(r, _C)
    xyzj = gx_ref[0].reshape(r, 4)[:, :3]
    p1 = p1_ref[0]
    p1r = jnp.broadcast_to(p1[None, :, :], (_K, bs, 3)).reshape(r, 3)
    d = xyzj - p1r
    t = _dot(d, m1_ref[...])
    h = jnp.maximum(t[:, :8] + c0_ref[...], 0.0)
    dirproj = t[:, 8:8 + _C]
    a1 = a1_ref[0]
    a1r = jnp.broadcast_to(a1[None, :, :], (_K, bs, _C)).reshape(r, _C)
    x = a1r + a2 + dirproj
    x = jnp.where(x >= 0.0, x, 0.1 * x)
    y = _dot(x, w1_ref[...]) + b1_ref[...]
    y = jnp.where(y >= 0.0, y, 0.1 * y)
    h = jnp.maximum(_dot(h, v1_ref[...]) + c1_ref[...], 0.0)
    w = jnp.maximum(_dot(h, v2_ref[...]) + c2_ref[...], 0.0)
    out_ref[0] = jnp.sum((w * y).reshape(_K, bs, _C), axis=0)


def _phase3(g1, g1x, a1rows, p1T, m1, c0, w1, b1, v1, c1, v2, c2):
    B, N1, _ = p1T.shape
    return pl.pallas_call(
        _phase3_body,
        grid=(B, N1 // _BS3),
        in_specs=[
            pl.BlockSpec((1, _K, _BS3, _C), lambda b, i: (b, 0, i, 0)),
            pl.BlockSpec((1, _K, _BS3, 4), lambda b, i: (b, 0, i, 0)),
            pl.BlockSpec((1, _BS3, _C), lambda b, i: (b, i, 0)),
            pl.BlockSpec((1, _BS3, 3), lambda b, i: (b, i, 0)),
            pl.BlockSpec((8 + _C, 3), lambda b, i: (0, 0)),
            pl.BlockSpec((1, 8), lambda b, i: (0, 0)),
            pl.BlockSpec((_C, _C), lambda b, i: (0, 0)),
            pl.BlockSpec((1, _C), lambda b, i: (0, 0)),
            pl.BlockSpec((8, 8), lambda b, i: (0, 0)),
            pl.BlockSpec((1, 8), lambda b, i: (0, 0)),
            pl.BlockSpec((_C, 8), lambda b, i: (0, 0)),
            pl.BlockSpec((1, _C), lambda b, i: (0, 0)),
        ],
        out_specs=pl.BlockSpec((1, _BS3, _C), lambda b, i: (b, i, 0)),
        out_shape=jax.ShapeDtypeStruct((B, N1, _C), jnp.float32),
    )(g1, g1x, a1rows, p1T, m1, c0, w1, b1, v1, c1, v2, c2)


def _phase5_body(g_ref, gx_ref, p1_ref, v0_ref, c0_ref, v1_ref, c1_ref,
                 v2_ref, c2_ref, out_ref):
    bs = p1_ref.shape[1]
    r = bs * _K
    nfj = g_ref[0].reshape(r, _C)
    xyzj = gx_ref[0].reshape(r, 4)[:, :3]
    p1 = p1_ref[0]
    p1r = jnp.broadcast_to(p1[None, :, :], (_K, bs, 3)).reshape(r, 3)
    d = xyzj - p1r
    h = jnp.maximum(_dot(d, v0_ref[...]) + c0_ref[...], 0.0)
    h = jnp.maximum(_dot(h, v1_ref[...]) + c1_ref[...], 0.0)
    w = jnp.maximum(_dot(h, v2_ref[...]) + c2_ref[...], 0.0)
    out_ref[0] = jnp.sum((w * nfj).reshape(_K, bs, _C), axis=0)


def _phase5(g2, g2x, p1T, v0, c0, v1, c1, v2, c2):
    B, N1, _ = p1T.shape
    return pl.pallas_call(
        _phase5_body,
        grid=(B, N1 // _BS3),
        in_specs=[
            pl.BlockSpec((1, _K, _BS3, _C), lambda b, i: (b, 0, i, 0)),
            pl.BlockSpec((1, _K, _BS3, 4), lambda b, i: (b, 0, i, 0)),
            pl.BlockSpec((1, _BS3, 3), lambda b, i: (b, i, 0)),
            pl.BlockSpec((8, 3), lambda b, i: (0, 0)),
            pl.BlockSpec((1, 8), lambda b, i: (0, 0)),
            pl.BlockSpec((8, 8), lambda b, i: (0, 0)),
            pl.BlockSpec((1, 8), lambda b, i: (0, 0)),
            pl.BlockSpec((_C, 8), lambda b, i: (0, 0)),
            pl.BlockSpec((1, _C), lambda b, i: (0, 0)),
        ],
        out_specs=pl.BlockSpec((1, _BS3, _C), lambda b, i: (b, i, 0)),
        out_shape=jax.ShapeDtypeStruct((B, N1, _C), jnp.float32),
    )(g2, g2x, p1T, v0, c0, v1, c1, v2, c2)


def kernel(pc1, pc2, feature1, feature2, mlp_W0, mlp_b0, mlp_W1, mlp_b1,
           wn1_W0, wn1_b0, wn1_W1, wn1_b1, wn1_W2, wn1_b2,
           wn2_W0, wn2_b0, wn2_W1, wn2_b1, wn2_W2, wn2_b2):
    B, _, N1 = pc1.shape
    N2 = pc2.shape[2]
    p1T = jnp.transpose(pc1, (0, 2, 1))
    w0a = mlp_W0[:, :_D]
    w0b = mlp_W0[:, _D:2 * _D]
    m1 = jnp.concatenate([wn1_W0, mlp_W0[:, 2 * _D:]], axis=0)

    p2T = jnp.transpose(pc2, (0, 2, 1))
    a1rows, t2, q1aug, k1aug, k2aug = _precompute(
        feature1, feature2, p1T, p2T, w0a, w0b, mlp_b0[None])
    total = B * N1 * _K
    p2rows = jnp.transpose(pc2, (1, 0, 2)).reshape(3, B * N2)
    p1rows = jnp.transpose(pc1, (1, 0, 2)).reshape(3, B * N1)

    idx1 = _knn(q1aug, k2aug)
    g1, g1xc = _sc_gather(t2.reshape(B * N2, _C),
                          p2rows[0], p2rows[1], p2rows[2], idx1.reshape(-1))
    idx2 = _knn(q1aug, k1aug)
    g1x = jnp.transpose(g1xc, (0, 2, 1)).reshape(total, 4)
    nf = _phase3(g1.reshape(B, _K, N1, _C), g1x.reshape(B, _K, N1, 4),
                 a1rows, p1T, m1,
                 wn1_b0[None], mlp_W1, mlp_b1[None],
                 wn1_W1, wn1_b1[None], wn1_W2, wn1_b2[None])

    g2, g2xc = _sc_gather(nf.reshape(B * N1, _C),
                          p1rows[0], p1rows[1], p1rows[2], idx2.reshape(-1))
    g2x = jnp.transpose(g2xc, (0, 2, 1)).reshape(total, 4)
    out_rows = _phase5(g2.reshape(B, _K, N1, _C), g2x.reshape(B, _K, N1, 4),
                       p1T,
                       wn2_W0, wn2_b0[None], wn2_W1, wn2_b1[None],
                       wn2_W2, wn2_b2[None])
    return jnp.transpose(out_rows, (0, 2, 1))


# speedup vs baseline: 1.5712x; 1.0164x over previous
import functools

import jax
import jax.numpy as jnp
from jax import lax
from jax.experimental import pallas as pl
from jax.experimental.pallas import tpu as pltpu
from jax.experimental.pallas import tpu_sc as plsc

_K = 16
_C = 128
_D = 64
_NB0 = 512
_BSQ = 512
_BS3 = 128


def _dot(x, w):
    return lax.dot_general(x, w, (((1,), (1,)), ((), ())),
                           preferred_element_type=jnp.float32)


def _precompute_body(f1_ref, f2_ref, p1_ref, p2_ref, w0a_ref, w0b_ref,
                     b0_ref, a1_ref, t2_ref, q1_ref, k1_ref, k2_ref):
    f1 = f1_ref[0]
    f2 = f2_ref[0]
    a1 = lax.dot_general(f1, w0a_ref[...], (((0,), (1,)), ((), ())),
                         preferred_element_type=jnp.float32)
    a2 = lax.dot_general(f2, w0b_ref[...], (((0,), (1,)), ((), ())),
                         preferred_element_type=jnp.float32)
    a1_ref[0] = a1 + b0_ref[...]
    t2_ref[0] = a2
    p1 = p1_ref[0]
    p2 = p2_ref[0]
    nb = p1.shape[0]
    one = jnp.ones((nb, 1), jnp.float32)
    zero = jnp.zeros((nb, 3), jnp.float32)
    p1sq = jnp.sum(p1 * p1, axis=1, keepdims=True)
    p2sq = jnp.sum(p2 * p2, axis=1, keepdims=True)
    q1_ref[0] = jnp.concatenate([-2.0 * p1, one, p1sq, zero], axis=1)
    k1_ref[0] = jnp.concatenate([p1, p1sq, one, zero], axis=1)
    k2_ref[0] = jnp.concatenate([p2, p2sq, one, zero], axis=1)


def _precompute(feature1, feature2, p1T, p2T, w0a, w0b, b0row):
    B, D, N = feature1.shape
    grid = (B, N // _NB0)
    aug = jax.ShapeDtypeStruct((B, N, 8), jnp.float32)
    return pl.pallas_call(
        _precompute_body,
        grid=grid,
        in_specs=[
            pl.BlockSpec((1, D, _NB0), lambda b, i: (b, 0, i)),
            pl.BlockSpec((1, D, _NB0), lambda b, i: (b, 0, i)),
            pl.BlockSpec((1, _NB0, 3), lambda b, i: (b, i, 0)),
            pl.BlockSpec((1, _NB0, 3), lambda b, i: (b, i, 0)),
            pl.BlockSpec((_C, D), lambda b, i: (0, 0)),
            pl.BlockSpec((_C, D), lambda b, i: (0, 0)),
            pl.BlockSpec((1, _C), lambda b, i: (0, 0)),
        ],
        out_specs=[
            pl.BlockSpec((1, _NB0, _C), lambda b, i: (b, i, 0)),
            pl.BlockSpec((1, _NB0, _C), lambda b, i: (b, i, 0)),
            pl.BlockSpec((1, _NB0, 8), lambda b, i: (b, i, 0)),
            pl.BlockSpec((1, _NB0, 8), lambda b, i: (b, i, 0)),
            pl.BlockSpec((1, _NB0, 8), lambda b, i: (b, i, 0)),
        ],
        out_shape=[
            jax.ShapeDtypeStruct((B, N, _C), jnp.float32),
            jax.ShapeDtypeStruct((B, N, _C), jnp.float32),
            aug, aug, aug,
        ],
    )(feature1, feature2, p1T, p2T, w0a, w0b, b0row)


_S = 4


def _ce(a, b):
    return jnp.minimum(a, b), jnp.maximum(a, b)


def _sort4(a, b, c, d):
    a, b = _ce(a, b)
    c, d = _ce(c, d)
    a, c = _ce(a, c)
    b, d = _ce(b, d)
    b, c = _ce(b, c)
    return [a, b, c, d]


def _merge4(x, y):
    c0 = jnp.minimum(x[0], y[3])
    c1 = jnp.minimum(x[1], y[2])
    c2 = jnp.minimum(x[2], y[1])
    c3 = jnp.minimum(x[3], y[0])
    c0, c2 = _ce(c0, c2)
    c1, c3 = _ce(c1, c3)
    c0, c1 = _ce(c0, c1)
    c2, c3 = _ce(c2, c3)
    return [c0, c1, c2, c3]


def _knn_body(q_ref, k_ref, idx_ref, *, n_keys):
    qa = q_ref[0]
    ka = k_ref[0]
    bs = qa.shape[0]
    ng = n_keys // 128
    kq = lax.dot_general(ka[:, :3], qa[:, :3], (((1,), (1,)), ((), ())),
                         preferred_element_type=jnp.float32)
    e4 = (lax.broadcasted_iota(jnp.int32, (1, 8), 1) == 4).astype(jnp.float32)
    qn = lax.dot_general(e4, qa, (((1,), (1,)), ((), ())),
                         preferred_element_type=jnp.float32)
    kn = ka[:, 3:4]
    distT = kq + kn + qn
    keys3 = (lax.bitcast_convert_type(distT, jnp.int32).reshape(ng, 128, bs)
             & jnp.int32(-ng)) | lax.broadcasted_iota(jnp.int32,
                                                      (ng, 128, bs), 0)
    maxi = jnp.int32(2 ** 31 - 1)
    big = jnp.int32(2 ** 30)
    quads = [_sort4(keys3[4 * i], keys3[4 * i + 1],
                    keys3[4 * i + 2], keys3[4 * i + 3])
             for i in range(ng // 4)]
    while len(quads) > 1:
        quads = [_merge4(quads[2 * i], quads[2 * i + 1])
                 for i in range(len(quads) // 2)]
    stack = quads[0]
    s_iota = lax.broadcasted_iota(jnp.int32, (128, bs), 0)
    colcur = stack[0]
    cnt = jnp.zeros((128, bs), jnp.int32)
    rows = []
    for _ in range(_K):
        m = jnp.min(colcur, axis=0, keepdims=True)
        sel = colcur == m
        sstar = jnp.min(jnp.where(sel, s_iota, big),
                        axis=0, keepdims=True)
        cstar = m & jnp.int32(ng - 1)
        rows.append(cstar * 128 + sstar)
        hit = s_iota == sstar
        cnt = cnt + jnp.where(hit, 1, 0)
        refill = jnp.full((128, bs), maxi, jnp.int32)
        for s in range(1, _S):
            refill = jnp.where(cnt == s, stack[s], refill)
        colcur = jnp.where(hit, refill, colcur)
    idx = jnp.concatenate(rows, axis=0)
    idx_ref[0] = idx + pl.program_id(0) * n_keys


def _knn(q_aug, k_aug):
    B, N1, _ = q_aug.shape
    N2 = k_aug.shape[1]
    return pl.pallas_call(
        functools.partial(_knn_body, n_keys=N2),
        grid=(B, N1 // _BSQ),
        in_specs=[
            pl.BlockSpec((1, _BSQ, 8), lambda b, i: (b, i, 0)),
            pl.BlockSpec((1, N2, 8), lambda b, i: (b, 0, 0)),
        ],
        out_specs=pl.BlockSpec((1, _K, _BSQ), lambda b, i: (b, 0, i)),
        out_shape=jax.ShapeDtypeStruct((B, _K, N1), jnp.int32),
    )(q_aug, k_aug)


def _sc_gather(table, xrow, yrow, zrow, idx):
    total = idx.shape[0]
    n_chunks = total // 128
    idx2d = idx.reshape(n_chunks, 128)
    per_w = n_chunks // 32
    npts = xrow.shape[0]
    mesh = plsc.VectorSubcoreMesh(core_axis_name="c", subcore_axis_name="s")

    @functools.partial(
        pl.kernel, mesh=mesh,
        compiler_params=pltpu.CompilerParams(needs_layout_passes=False),
        out_type=[
            jax.ShapeDtypeStruct((total, _C), jnp.float32),
            jax.ShapeDtypeStruct((n_chunks, 4, 128), jnp.float32),
        ],
        scratch_types=[
            pltpu.VMEM((128,), jnp.int32),
            pltpu.VMEM((128,), jnp.int32),
            pltpu.VMEM((128, _C), jnp.float32),
            pltpu.VMEM((128, _C), jnp.float32),
            pltpu.VMEM((4, 128), jnp.float32),
            pltpu.VMEM((4, 128), jnp.float32),
            pltpu.VMEM((npts,), jnp.float32),
            pltpu.VMEM((npts,), jnp.float32),
            pltpu.VMEM((npts,), jnp.float32),
            pltpu.SemaphoreType.DMA,
            pltpu.SemaphoreType.DMA,
        ],
    )
    def gk(table_hbm, x_hbm, y_hbm, z_hbm, idx_hbm, out_hbm, xyz_hbm,
           idxv0, idxv1, rows0, rows1, xyz0, xyz1, xv, yv, zv, sem0, sem1):
        wid = lax.axis_index("s") * 2 + lax.axis_index("c")
        base = wid * per_w
        pltpu.sync_copy(x_hbm, xv)
        pltpu.sync_copy(y_hbm, yv)
        pltpu.sync_copy(z_hbm, zv)

        def start(row, idxv, rows, sem):
            pltpu.sync_copy(idx_hbm.at[row], idxv)
            pltpu.async_copy(table_hbm.at[idxv], rows, sem)

        def finish(row, idxv, rows, xyzbuf, sem):
            pltpu.make_async_copy(table_hbm.at[idxv], rows, sem).wait()
            for g in range(8):
                iv = idxv[pl.ds(g * 16, 16)]
                xyzbuf[0, pl.ds(g * 16, 16)] = plsc.load_gather(xv, [iv])
                xyzbuf[1, pl.ds(g * 16, 16)] = plsc.load_gather(yv, [iv])
                xyzbuf[2, pl.ds(g * 16, 16)] = plsc.load_gather(zv, [iv])
            pltpu.sync_copy(rows, out_hbm.at[pl.ds(row * 128, 128)])
            pltpu.sync_copy(xyzbuf, xyz_hbm.at[row])

        start(base, idxv0, rows0, sem0)
        start(base + 1, idxv1, rows1, sem1)

        def body(i, carry):
            row = base + 2 * i
            finish(row, idxv0, rows0, xyz0, sem0)
            start(row + 2, idxv0, rows0, sem0)
            finish(row + 1, idxv1, rows1, xyz1, sem1)
            start(row + 3, idxv1, rows1, sem1)
            return carry

        lax.fori_loop(0, per_w // 2 - 1, body, 0)
        last = base + per_w - 2
        finish(last, idxv0, rows0, xyz0, sem0)
        finish(last + 1, idxv1, rows1, xyz1, sem1)

    return gk(table, xrow, yrow, zrow, idx2d)


def _phase3_body(g_ref, gx_ref, a1_ref, p1_ref, m1_ref, c0_ref, w1_ref,
                 b1_ref, v1_ref, c1_ref, v2_ref, c2_ref, out_ref):
    bs = p1_ref.shape[1]
    r = bs * _K
    a2 = g_ref[0].reshape(r, _C)
    xyzj = gx_ref[0].reshape(r, 4)[:, :3]
    p1 = p1_ref[0]
    p1r = jnp.broadcast_to(p1[None, :, :], (_K, bs, 3)).reshape(r, 3)
    d = xyzj - p1r
    t = _dot(d, m1_ref[...])
    h = jnp.maximum(t[:, :8] + c0_ref[...], 0.0)
    dirproj = t[:, 8:8 + _C]
    a1 = a1_ref[0]
    a1r = jnp.broadcast_to(a1[None, :, :], (_K, bs, _C)).reshape(r, _C)
    x = a1r + a2 + dirproj
    x = jnp.where(x >= 0.0, x, 0.1 * x)
    y = _dot(x, w1_ref[...]) + b1_ref[...]
    y = jnp.where(y >= 0.0, y, 0.1 * y)
    h = jnp.maximum(_dot(h, v1_ref[...]) + c1_ref[...], 0.0)
    w = jnp.maximum(_dot(h, v2_ref[...]) + c2_ref[...], 0.0)
    out_ref[0] = jnp.sum((w * y).reshape(_K, bs, _C), axis=0)


def _phase3(g1, g1x, a1rows, p1T, m1, c0, w1, b1, v1, c1, v2, c2):
    B, N1, _ = p1T.shape
    return pl.pallas_call(
        _phase3_body,
        grid=(B, N1 // _BS3),
        in_specs=[
            pl.BlockSpec((1, _K, _BS3, _C), lambda b, i: (b, 0, i, 0)),
            pl.BlockSpec((1, _K, _BS3, 4), lambda b, i: (b, 0, i, 0)),
            pl.BlockSpec((1, _BS3, _C), lambda b, i: (b, i, 0)),
            pl.BlockSpec((1, _BS3, 3), lambda b, i: (b, i, 0)),
            pl.BlockSpec((8 + _C, 3), lambda b, i: (0, 0)),
            pl.BlockSpec((1, 8), lambda b, i: (0, 0)),
            pl.BlockSpec((_C, _C), lambda b, i: (0, 0)),
            pl.BlockSpec((1, _C), lambda b, i: (0, 0)),
            pl.BlockSpec((8, 8), lambda b, i: (0, 0)),
            pl.BlockSpec((1, 8), lambda b, i: (0, 0)),
            pl.BlockSpec((_C, 8), lambda b, i: (0, 0)),
            pl.BlockSpec((1, _C), lambda b, i: (0, 0)),
        ],
        out_specs=pl.BlockSpec((1, _BS3, _C), lambda b, i: (b, i, 0)),
        out_shape=jax.ShapeDtypeStruct((B, N1, _C), jnp.float32),
    )(g1, g1x, a1rows, p1T, m1, c0, w1, b1, v1, c1, v2, c2)


def _phase5_body(g_ref, gx_ref, p1_ref, v0_ref, c0_ref, v1_ref, c1_ref,
                 v2_ref, c2_ref, out_ref):
    bs = p1_ref.shape[1]
    r = bs * _K
    nfj = g_ref[0].reshape(r, _C)
    xyzj = gx_ref[0].reshape(r, 4)[:, :3]
    p1 = p1_ref[0]
    p1r = jnp.broadcast_to(p1[None, :, :], (_K, bs, 3)).reshape(r, 3)
    d = xyzj - p1r
    h = jnp.maximum(_dot(d, v0_ref[...]) + c0_ref[...], 0.0)
    h = jnp.maximum(_dot(h, v1_ref[...]) + c1_ref[...], 0.0)
    w = jnp.maximum(_dot(h, v2_ref[...]) + c2_ref[...], 0.0)
    out_ref[0] = jnp.sum((w * nfj).reshape(_K, bs, _C), axis=0)


def _phase5(g2, g2x, p1T, v0, c0, v1, c1, v2, c2):
    B, N1, _ = p1T.shape
    return pl.pallas_call(
        _phase5_body,
        grid=(B, N1 // _BS3),
        in_specs=[
            pl.BlockSpec((1, _K, _BS3, _C), lambda b, i: (b, 0, i, 0)),
            pl.BlockSpec((1, _K, _BS3, 4), lambda b, i: (b, 0, i, 0)),
            pl.BlockSpec((1, _BS3, 3), lambda b, i: (b, i, 0)),
            pl.BlockSpec((8, 3), lambda b, i: (0, 0)),
            pl.BlockSpec((1, 8), lambda b, i: (0, 0)),
            pl.BlockSpec((8, 8), lambda b, i: (0, 0)),
            pl.BlockSpec((1, 8), lambda b, i: (0, 0)),
            pl.BlockSpec((_C, 8), lambda b, i: (0, 0)),
            pl.BlockSpec((1, _C), lambda b, i: (0, 0)),
        ],
        out_specs=pl.BlockSpec((1, _BS3, _C), lambda b, i: (b, i, 0)),
        out_shape=jax.ShapeDtypeStruct((B, N1, _C), jnp.float32),
    )(g2, g2x, p1T, v0, c0, v1, c1, v2, c2)


def kernel(pc1, pc2, feature1, feature2, mlp_W0, mlp_b0, mlp_W1, mlp_b1,
           wn1_W0, wn1_b0, wn1_W1, wn1_b1, wn1_W2, wn1_b2,
           wn2_W0, wn2_b0, wn2_W1, wn2_b1, wn2_W2, wn2_b2):
    B, _, N1 = pc1.shape
    N2 = pc2.shape[2]
    p1T = jnp.transpose(pc1, (0, 2, 1))
    w0a = mlp_W0[:, :_D]
    w0b = mlp_W0[:, _D:2 * _D]
    m1 = jnp.concatenate([wn1_W0, mlp_W0[:, 2 * _D:]], axis=0)

    p2T = jnp.transpose(pc2, (0, 2, 1))
    a1rows, t2, q1aug, k1aug, k2aug = _precompute(
        feature1, feature2, p1T, p2T, w0a, w0b, mlp_b0[None])
    total = B * N1 * _K
    p2rows = jnp.transpose(pc2, (1, 0, 2)).reshape(3, B * N2)
    p1rows = jnp.transpose(pc1, (1, 0, 2)).reshape(3, B * N1)

    idx1 = _knn(q1aug, k2aug)
    g1, g1xc = _sc_gather(t2.reshape(B * N2, _C),
                          p2rows[0], p2rows[1], p2rows[2], idx1.reshape(-1))
    idx2 = _knn(q1aug, k1aug)
    g1x = jnp.transpose(g1xc, (0, 2, 1)).reshape(total, 4)
    nf = _phase3(g1.reshape(B, _K, N1, _C), g1x.reshape(B, _K, N1, 4),
                 a1rows, p1T, m1,
                 wn1_b0[None], mlp_W1, mlp_b1[None],
                 wn1_W1, wn1_b1[None], wn1_W2, wn1_b2[None])

    g2, g2xc = _sc_gather(nf.reshape(B * N1, _C),
                          p1rows[0], p1rows[1], p1rows[2], idx2.reshape(-1))
    g2x = jnp.transpose(g2xc, (0, 2, 1)).reshape(total, 4)
    out_rows = _phase5(g2.reshape(B, _K, N1, _C), g2x.reshape(B, _K, N1, 4),
                       p1T,
                       wn2_W0, wn2_b0[None], wn2_W1, wn2_b1[None],
                       wn2_W2, wn2_b2[None])
    return jnp.transpose(out_rows, (0, 2, 1))


# speedup vs baseline: 1.6845x; 1.0721x over previous
import functools

import jax
import jax.numpy as jnp
from jax import lax
from jax.experimental import pallas as pl
from jax.experimental.pallas import tpu as pltpu
from jax.experimental.pallas import tpu_sc as plsc

_K = 16
_C = 128
_D = 64
_NB0 = 512
_BSQ = 512
_BS3 = 256


def _dot(x, w):
    return lax.dot_general(x, w, (((1,), (1,)), ((), ())),
                           preferred_element_type=jnp.float32)


def _precompute_body(f1_ref, f2_ref, p1_ref, p2_ref, w0a_ref, w0b_ref,
                     b0_ref, a1_ref, t2_ref, q1_ref, k1_ref, k2_ref):
    f1 = f1_ref[0]
    f2 = f2_ref[0]
    a1 = lax.dot_general(f1, w0a_ref[...], (((0,), (1,)), ((), ())),
                         preferred_element_type=jnp.float32)
    a2 = lax.dot_general(f2, w0b_ref[...], (((0,), (1,)), ((), ())),
                         preferred_element_type=jnp.float32)
    a1_ref[0] = a1 + b0_ref[...]
    t2_ref[0] = a2
    p1 = p1_ref[0]
    p2 = p2_ref[0]
    nb = p1.shape[0]
    one = jnp.ones((nb, 1), jnp.float32)
    zero = jnp.zeros((nb, 3), jnp.float32)
    p1sq = jnp.sum(p1 * p1, axis=1, keepdims=True)
    p2sq = jnp.sum(p2 * p2, axis=1, keepdims=True)
    q1_ref[0] = jnp.concatenate([-2.0 * p1, one, p1sq, zero], axis=1)
    k1_ref[0] = jnp.concatenate([p1, p1sq, one, zero], axis=1)
    k2_ref[0] = jnp.concatenate([p2, p2sq, one, zero], axis=1)


def _precompute(feature1, feature2, p1T, p2T, w0a, w0b, b0row):
    B, D, N = feature1.shape
    grid = (B, N // _NB0)
    aug = jax.ShapeDtypeStruct((B, N, 8), jnp.float32)
    return pl.pallas_call(
        _precompute_body,
        grid=grid,
        in_specs=[
            pl.BlockSpec((1, D, _NB0), lambda b, i: (b, 0, i)),
            pl.BlockSpec((1, D, _NB0), lambda b, i: (b, 0, i)),
            pl.BlockSpec((1, _NB0, 3), lambda b, i: (b, i, 0)),
            pl.BlockSpec((1, _NB0, 3), lambda b, i: (b, i, 0)),
            pl.BlockSpec((_C, D), lambda b, i: (0, 0)),
            pl.BlockSpec((_C, D), lambda b, i: (0, 0)),
            pl.BlockSpec((1, _C), lambda b, i: (0, 0)),
        ],
        out_specs=[
            pl.BlockSpec((1, _NB0, _C), lambda b, i: (b, i, 0)),
            pl.BlockSpec((1, _NB0, _C), lambda b, i: (b, i, 0)),
            pl.BlockSpec((1, _NB0, 8), lambda b, i: (b, i, 0)),
            pl.BlockSpec((1, _NB0, 8), lambda b, i: (b, i, 0)),
            pl.BlockSpec((1, _NB0, 8), lambda b, i: (b, i, 0)),
        ],
        out_shape=[
            jax.ShapeDtypeStruct((B, N, _C), jnp.float32),
            jax.ShapeDtypeStruct((B, N, _C), jnp.float32),
            aug, aug, aug,
        ],
    )(feature1, feature2, p1T, p2T, w0a, w0b, b0row)


_S = 4


def _ce(a, b):
    return jnp.minimum(a, b), jnp.maximum(a, b)


def _sort4(a, b, c, d):
    a, b = _ce(a, b)
    c, d = _ce(c, d)
    a, c = _ce(a, c)
    b, d = _ce(b, d)
    b, c = _ce(b, c)
    return [a, b, c, d]


def _merge4(x, y):
    c0 = jnp.minimum(x[0], y[3])
    c1 = jnp.minimum(x[1], y[2])
    c2 = jnp.minimum(x[2], y[1])
    c3 = jnp.minimum(x[3], y[0])
    c0, c2 = _ce(c0, c2)
    c1, c3 = _ce(c1, c3)
    c0, c1 = _ce(c0, c1)
    c2, c3 = _ce(c2, c3)
    return [c0, c1, c2, c3]


def _knn_body(q_ref, k_ref, idx_ref, *, n_keys):
    qa = q_ref[0]
    ka = k_ref[0]
    bs = qa.shape[0]
    ng = n_keys // 128
    kq = lax.dot_general(ka[:, :3], qa[:, :3], (((1,), (1,)), ((), ())),
                         preferred_element_type=jnp.float32)
    e4 = (lax.broadcasted_iota(jnp.int32, (1, 8), 1) == 4).astype(jnp.float32)
    qn = lax.dot_general(e4, qa, (((1,), (1,)), ((), ())),
                         preferred_element_type=jnp.float32)
    kn = ka[:, 3:4]
    distT = kq + kn + qn
    keys3 = (lax.bitcast_convert_type(distT, jnp.int32).reshape(ng, 128, bs)
             & jnp.int32(-ng)) | lax.broadcasted_iota(jnp.int32,
                                                      (ng, 128, bs), 0)
    maxi = jnp.int32(2 ** 31 - 1)
    big = jnp.int32(2 ** 30)
    quads = [_sort4(keys3[4 * i], keys3[4 * i + 1],
                    keys3[4 * i + 2], keys3[4 * i + 3])
             for i in range(ng // 4)]
    while len(quads) > 1:
        quads = [_merge4(quads[2 * i], quads[2 * i + 1])
                 for i in range(len(quads) // 2)]
    stack = quads[0]
    s_iota = lax.broadcasted_iota(jnp.int32, (128, bs), 0)
    colcur = stack[0]
    cnt = jnp.zeros((128, bs), jnp.int32)
    rows = []
    for _ in range(_K):
        m = jnp.min(colcur, axis=0, keepdims=True)
        sel = colcur == m
        sstar = jnp.min(jnp.where(sel, s_iota, big),
                        axis=0, keepdims=True)
        cstar = m & jnp.int32(ng - 1)
        rows.append(cstar * 128 + sstar)
        hit = s_iota == sstar
        cnt = cnt + jnp.where(hit, 1, 0)
        refill = jnp.full((128, bs), maxi, jnp.int32)
        for s in range(1, _S):
            refill = jnp.where(cnt == s, stack[s], refill)
        colcur = jnp.where(hit, refill, colcur)
    idx = jnp.concatenate(rows, axis=0)
    idx_ref[0] = idx + pl.program_id(0) * n_keys


def _knn(q_aug, k_aug):
    B, N1, _ = q_aug.shape
    N2 = k_aug.shape[1]
    return pl.pallas_call(
        functools.partial(_knn_body, n_keys=N2),
        grid=(B, N1 // _BSQ),
        in_specs=[
            pl.BlockSpec((1, _BSQ, 8), lambda b, i: (b, i, 0)),
            pl.BlockSpec((1, N2, 8), lambda b, i: (b, 0, 0)),
        ],
        out_specs=pl.BlockSpec((1, _K, _BSQ), lambda b, i: (b, 0, i)),
        out_shape=jax.ShapeDtypeStruct((B, _K, N1), jnp.int32),
    )(q_aug, k_aug)


def _sc_gather(table, xrow, yrow, zrow, idx):
    total = idx.shape[0]
    n_chunks = total // 128
    idx2d = idx.reshape(n_chunks, 128)
    per_w = n_chunks // 32
    npts = xrow.shape[0]
    mesh = plsc.VectorSubcoreMesh(core_axis_name="c", subcore_axis_name="s")

    @functools.partial(
        pl.kernel, mesh=mesh,
        compiler_params=pltpu.CompilerParams(needs_layout_passes=False),
        out_type=[
            jax.ShapeDtypeStruct((total, _C), jnp.float32),
            jax.ShapeDtypeStruct((n_chunks, 4, 128), jnp.float32),
        ],
        scratch_types=[
            pltpu.VMEM((128,), jnp.int32),
            pltpu.VMEM((128,), jnp.int32),
            pltpu.VMEM((128, _C), jnp.float32),
            pltpu.VMEM((128, _C), jnp.float32),
            pltpu.VMEM((4, 128), jnp.float32),
            pltpu.VMEM((4, 128), jnp.float32),
            pltpu.VMEM((npts,), jnp.float32),
            pltpu.VMEM((npts,), jnp.float32),
            pltpu.VMEM((npts,), jnp.float32),
            pltpu.SemaphoreType.DMA,
            pltpu.SemaphoreType.DMA,
        ],
    )
    def gk(table_hbm, x_hbm, y_hbm, z_hbm, idx_hbm, out_hbm, xyz_hbm,
           idxv0, idxv1, rows0, rows1, xyz0, xyz1, xv, yv, zv, sem0, sem1):
        wid = lax.axis_index("s") * 2 + lax.axis_index("c")
        base = wid * per_w
        pltpu.sync_copy(x_hbm, xv)
        pltpu.sync_copy(y_hbm, yv)
        pltpu.sync_copy(z_hbm, zv)

        def start(row, idxv, rows, sem):
            pltpu.sync_copy(idx_hbm.at[row], idxv)
            pltpu.async_copy(table_hbm.at[idxv], rows, sem)

        def finish(row, idxv, rows, xyzbuf, sem):
            pltpu.make_async_copy(table_hbm.at[idxv], rows, sem).wait()
            for g in range(8):
                iv = idxv[pl.ds(g * 16, 16)]
                xyzbuf[0, pl.ds(g * 16, 16)] = plsc.load_gather(xv, [iv])
                xyzbuf[1, pl.ds(g * 16, 16)] = plsc.load_gather(yv, [iv])
                xyzbuf[2, pl.ds(g * 16, 16)] = plsc.load_gather(zv, [iv])
            pltpu.sync_copy(rows, out_hbm.at[pl.ds(row * 128, 128)])
            pltpu.sync_copy(xyzbuf, xyz_hbm.at[row])

        start(base, idxv0, rows0, sem0)
        start(base + 1, idxv1, rows1, sem1)

        def body(i, carry):
            row = base + 2 * i
            finish(row, idxv0, rows0, xyz0, sem0)
            start(row + 2, idxv0, rows0, sem0)
            finish(row + 1, idxv1, rows1, xyz1, sem1)
            start(row + 3, idxv1, rows1, sem1)
            return carry

        lax.fori_loop(0, per_w // 2 - 1, body, 0)
        last = base + per_w - 2
        finish(last, idxv0, rows0, xyz0, sem0)
        finish(last + 1, idxv1, rows1, xyz1, sem1)

    return gk(table, xrow, yrow, zrow, idx2d)


def _phase3_body(g_ref, gx_ref, a1_ref, p1_ref, m1_ref, c0_ref, w1_ref,
                 b1_ref, v1_ref, c1_ref, v2_ref, c2_ref, out_ref):
    bs = p1_ref.shape[1]
    r = bs * _K
    a2 = g_ref[0].reshape(r, _C)
    xyzj = gx_ref[0].reshape(r, 4)[:, :3]
    p1 = p1_ref[0]
    p1r = jnp.broadcast_to(p1[None, :, :], (_K, bs, 3)).reshape(r, 3)
    d = xyzj - p1r
    t = _dot(d, m1_ref[...])
    h = jnp.maximum(t[:, :8] + c0_ref[...], 0.0)
    dirproj = t[:, 8:8 + _C]
    a1 = a1_ref[0]
    a1r = jnp.broadcast_to(a1[None, :, :], (_K, bs, _C)).reshape(r, _C)
    x = a1r + a2 + dirproj
    x = jnp.where(x >= 0.0, x, 0.1 * x)
    y = _dot(x, w1_ref[...]) + b1_ref[...]
    y = jnp.where(y >= 0.0, y, 0.1 * y)
    h = jnp.maximum(_dot(h, v1_ref[...]) + c1_ref[...], 0.0)
    w = jnp.maximum(_dot(h, v2_ref[...]) + c2_ref[...], 0.0)
    out_ref[0] = jnp.sum((w * y).reshape(_K, bs, _C), axis=0)


def _phase3(g1, g1x, a1rows, p1T, m1, c0, w1, b1, v1, c1, v2, c2):
    B, N1, _ = p1T.shape
    return pl.pallas_call(
        _phase3_body,
        grid=(B, N1 // _BS3),
        in_specs=[
            pl.BlockSpec((1, _K, _BS3, _C), lambda b, i: (b, 0, i, 0)),
            pl.BlockSpec((1, _K, _BS3, 4), lambda b, i: (b, 0, i, 0)),
            pl.BlockSpec((1, _BS3, _C), lambda b, i: (b, i, 0)),
            pl.BlockSpec((1, _BS3, 3), lambda b, i: (b, i, 0)),
            pl.BlockSpec((8 + _C, 3), lambda b, i: (0, 0)),
            pl.BlockSpec((1, 8), lambda b, i: (0, 0)),
            pl.BlockSpec((_C, _C), lambda b, i: (0, 0)),
            pl.BlockSpec((1, _C), lambda b, i: (0, 0)),
            pl.BlockSpec((8, 8), lambda b, i: (0, 0)),
            pl.BlockSpec((1, 8), lambda b, i: (0, 0)),
            pl.BlockSpec((_C, 8), lambda b, i: (0, 0)),
            pl.BlockSpec((1, _C), lambda b, i: (0, 0)),
        ],
        out_specs=pl.BlockSpec((1, _BS3, _C), lambda b, i: (b, i, 0)),
        out_shape=jax.ShapeDtypeStruct((B, N1, _C), jnp.float32),
    )(g1, g1x, a1rows, p1T, m1, c0, w1, b1, v1, c1, v2, c2)


def _phase5_body(g_ref, gx_ref, p1_ref, v0_ref, c0_ref, v1_ref, c1_ref,
                 v2_ref, c2_ref, out_ref):
    bs = p1_ref.shape[1]
    r = bs * _K
    nfj = g_ref[0].reshape(r, _C)
    xyzj = gx_ref[0].reshape(r, 4)[:, :3]
    p1 = p1_ref[0]
    p1r = jnp.broadcast_to(p1[None, :, :], (_K, bs, 3)).reshape(r, 3)
    d = xyzj - p1r
    h = jnp.maximum(_dot(d, v0_ref[...]) + c0_ref[...], 0.0)
    h = jnp.maximum(_dot(h, v1_ref[...]) + c1_ref[...], 0.0)
    w = jnp.maximum(_dot(h, v2_ref[...]) + c2_ref[...], 0.0)
    out_ref[0] = jnp.sum((w * nfj).reshape(_K, bs, _C), axis=0)


def _phase5(g2, g2x, p1T, v0, c0, v1, c1, v2, c2):
    B, N1, _ = p1T.shape
    return pl.pallas_call(
        _phase5_body,
        grid=(B, N1 // _BS3),
        in_specs=[
            pl.BlockSpec((1, _K, _BS3, _C), lambda b, i: (b, 0, i, 0)),
            pl.BlockSpec((1, _K, _BS3, 4), lambda b, i: (b, 0, i, 0)),
            pl.BlockSpec((1, _BS3, 3), lambda b, i: (b, i, 0)),
            pl.BlockSpec((8, 3), lambda b, i: (0, 0)),
            pl.BlockSpec((1, 8), lambda b, i: (0, 0)),
            pl.BlockSpec((8, 8), lambda b, i: (0, 0)),
            pl.BlockSpec((1, 8), lambda b, i: (0, 0)),
            pl.BlockSpec((_C, 8), lambda b, i: (0, 0)),
            pl.BlockSpec((1, _C), lambda b, i: (0, 0)),
        ],
        out_specs=pl.BlockSpec((1, _BS3, _C), lambda b, i: (b, i, 0)),
        out_shape=jax.ShapeDtypeStruct((B, N1, _C), jnp.float32),
    )(g2, g2x, p1T, v0, c0, v1, c1, v2, c2)


def kernel(pc1, pc2, feature1, feature2, mlp_W0, mlp_b0, mlp_W1, mlp_b1,
           wn1_W0, wn1_b0, wn1_W1, wn1_b1, wn1_W2, wn1_b2,
           wn2_W0, wn2_b0, wn2_W1, wn2_b1, wn2_W2, wn2_b2):
    B, _, N1 = pc1.shape
    N2 = pc2.shape[2]
    p1T = jnp.transpose(pc1, (0, 2, 1))
    w0a = mlp_W0[:, :_D]
    w0b = mlp_W0[:, _D:2 * _D]
    m1 = jnp.concatenate([wn1_W0, mlp_W0[:, 2 * _D:]], axis=0)

    p2T = jnp.transpose(pc2, (0, 2, 1))
    a1rows, t2, q1aug, k1aug, k2aug = _precompute(
        feature1, feature2, p1T, p2T, w0a, w0b, mlp_b0[None])
    total = B * N1 * _K
    p2rows = jnp.transpose(pc2, (1, 0, 2)).reshape(3, B * N2)
    p1rows = jnp.transpose(pc1, (1, 0, 2)).reshape(3, B * N1)

    idx1 = _knn(q1aug, k2aug)
    g1, g1xc = _sc_gather(t2.reshape(B * N2, _C),
                          p2rows[0], p2rows[1], p2rows[2], idx1.reshape(-1))
    idx2 = _knn(q1aug, k1aug)
    g1x = jnp.transpose(g1xc, (0, 2, 1)).reshape(total, 4)
    nf = _phase3(g1.reshape(B, _K, N1, _C), g1x.reshape(B, _K, N1, 4),
                 a1rows, p1T, m1,
                 wn1_b0[None], mlp_W1, mlp_b1[None],
                 wn1_W1, wn1_b1[None], wn1_W2, wn1_b2[None])

    g2, g2xc = _sc_gather(nf.reshape(B * N1, _C),
                          p1rows[0], p1rows[1], p1rows[2], idx2.reshape(-1))
    g2x = jnp.transpose(g2xc, (0, 2, 1)).reshape(total, 4)
    out_rows = _phase5(g2.reshape(B, _K, N1, _C), g2x.reshape(B, _K, N1, 4),
                       p1T,
                       wn2_W0, wn2_b0[None], wn2_W1, wn2_b1[None],
                       wn2_W2, wn2_b2[None])
    return jnp.transpose(out_rows, (0, 2, 1))


# speedup vs baseline: 1.7330x; 1.0288x over previous
import functools

import jax
import jax.numpy as jnp
from jax import lax
from jax.experimental import pallas as pl
from jax.experimental.pallas import tpu as pltpu
from jax.experimental.pallas import tpu_sc as plsc

_K = 16
_C = 128
_D = 64
_NB0 = 512
_BSQ = 512
_BS3 = 512


def _dot(x, w):
    return lax.dot_general(x, w, (((1,), (1,)), ((), ())),
                           preferred_element_type=jnp.float32)


def _precompute_body(f1_ref, f2_ref, p1_ref, p2_ref, w0a_ref, w0b_ref,
                     b0_ref, a1_ref, t2_ref, q1_ref, k1_ref, k2_ref):
    f1 = f1_ref[0]
    f2 = f2_ref[0]
    a1 = lax.dot_general(f1, w0a_ref[...], (((0,), (1,)), ((), ())),
                         preferred_element_type=jnp.float32)
    a2 = lax.dot_general(f2, w0b_ref[...], (((0,), (1,)), ((), ())),
                         preferred_element_type=jnp.float32)
    a1_ref[0] = a1 + b0_ref[...]
    t2_ref[0] = a2
    p1 = p1_ref[0]
    p2 = p2_ref[0]
    nb = p1.shape[0]
    one = jnp.ones((nb, 1), jnp.float32)
    zero = jnp.zeros((nb, 3), jnp.float32)
    p1sq = jnp.sum(p1 * p1, axis=1, keepdims=True)
    p2sq = jnp.sum(p2 * p2, axis=1, keepdims=True)
    q1_ref[0] = jnp.concatenate([-2.0 * p1, one, p1sq, zero], axis=1)
    k1_ref[0] = jnp.concatenate([p1, p1sq, one, zero], axis=1)
    k2_ref[0] = jnp.concatenate([p2, p2sq, one, zero], axis=1)


def _precompute(feature1, feature2, p1T, p2T, w0a, w0b, b0row):
    B, D, N = feature1.shape
    grid = (B, N // _NB0)
    aug = jax.ShapeDtypeStruct((B, N, 8), jnp.float32)
    return pl.pallas_call(
        _precompute_body,
        grid=grid,
        in_specs=[
            pl.BlockSpec((1, D, _NB0), lambda b, i: (b, 0, i)),
            pl.BlockSpec((1, D, _NB0), lambda b, i: (b, 0, i)),
            pl.BlockSpec((1, _NB0, 3), lambda b, i: (b, i, 0)),
            pl.BlockSpec((1, _NB0, 3), lambda b, i: (b, i, 0)),
            pl.BlockSpec((_C, D), lambda b, i: (0, 0)),
            pl.BlockSpec((_C, D), lambda b, i: (0, 0)),
            pl.BlockSpec((1, _C), lambda b, i: (0, 0)),
        ],
        out_specs=[
            pl.BlockSpec((1, _NB0, _C), lambda b, i: (b, i, 0)),
            pl.BlockSpec((1, _NB0, _C), lambda b, i: (b, i, 0)),
            pl.BlockSpec((1, _NB0, 8), lambda b, i: (b, i, 0)),
            pl.BlockSpec((1, _NB0, 8), lambda b, i: (b, i, 0)),
            pl.BlockSpec((1, _NB0, 8), lambda b, i: (b, i, 0)),
        ],
        out_shape=[
            jax.ShapeDtypeStruct((B, N, _C), jnp.float32),
            jax.ShapeDtypeStruct((B, N, _C), jnp.float32),
            aug, aug, aug,
        ],
    )(feature1, feature2, p1T, p2T, w0a, w0b, b0row)


_S = 4


def _ce(a, b):
    return jnp.minimum(a, b), jnp.maximum(a, b)


def _sort4(a, b, c, d):
    a, b = _ce(a, b)
    c, d = _ce(c, d)
    a, c = _ce(a, c)
    b, d = _ce(b, d)
    b, c = _ce(b, c)
    return [a, b, c, d]


def _merge4(x, y):
    c0 = jnp.minimum(x[0], y[3])
    c1 = jnp.minimum(x[1], y[2])
    c2 = jnp.minimum(x[2], y[1])
    c3 = jnp.minimum(x[3], y[0])
    c0, c2 = _ce(c0, c2)
    c1, c3 = _ce(c1, c3)
    c0, c1 = _ce(c0, c1)
    c2, c3 = _ce(c2, c3)
    return [c0, c1, c2, c3]


def _knn_body(q_ref, k_ref, idx_ref, *, n_keys):
    qa = q_ref[0]
    ka = k_ref[0]
    bs = qa.shape[0]
    ng = n_keys // 128
    kq = lax.dot_general(ka[:, :3], qa[:, :3], (((1,), (1,)), ((), ())),
                         preferred_element_type=jnp.float32)
    e4 = (lax.broadcasted_iota(jnp.int32, (1, 8), 1) == 4).astype(jnp.float32)
    qn = lax.dot_general(e4, qa, (((1,), (1,)), ((), ())),
                         preferred_element_type=jnp.float32)
    kn = ka[:, 3:4]
    distT = kq + kn + qn
    keys3 = (lax.bitcast_convert_type(distT, jnp.int32).reshape(ng, 128, bs)
             & jnp.int32(-ng)) | lax.broadcasted_iota(jnp.int32,
                                                      (ng, 128, bs), 0)
    maxi = jnp.int32(2 ** 31 - 1)
    big = jnp.int32(2 ** 30)
    quads = [_sort4(keys3[4 * i], keys3[4 * i + 1],
                    keys3[4 * i + 2], keys3[4 * i + 3])
             for i in range(ng // 4)]
    while len(quads) > 1:
        quads = [_merge4(quads[2 * i], quads[2 * i + 1])
                 for i in range(len(quads) // 2)]
    stack = quads[0]
    s_iota = lax.broadcasted_iota(jnp.int32, (128, bs), 0)
    colcur = stack[0]
    cnt = jnp.zeros((128, bs), jnp.int32)
    rows = []
    for _ in range(_K):
        m = jnp.min(colcur, axis=0, keepdims=True)
        sel = colcur == m
        sstar = jnp.min(jnp.where(sel, s_iota, big),
                        axis=0, keepdims=True)
        cstar = m & jnp.int32(ng - 1)
        rows.append(cstar * 128 + sstar)
        hit = s_iota == sstar
        cnt = cnt + jnp.where(hit, 1, 0)
        refill = jnp.full((128, bs), maxi, jnp.int32)
        for s in range(1, _S):
            refill = jnp.where(cnt == s, stack[s], refill)
        colcur = jnp.where(hit, refill, colcur)
    idx = jnp.concatenate(rows, axis=0)
    idx_ref[0] = idx + pl.program_id(0) * n_keys


def _knn(q_aug, k_aug):
    B, N1, _ = q_aug.shape
    N2 = k_aug.shape[1]
    return pl.pallas_call(
        functools.partial(_knn_body, n_keys=N2),
        grid=(B, N1 // _BSQ),
        in_specs=[
            pl.BlockSpec((1, _BSQ, 8), lambda b, i: (b, i, 0)),
            pl.BlockSpec((1, N2, 8), lambda b, i: (b, 0, 0)),
        ],
        out_specs=pl.BlockSpec((1, _K, _BSQ), lambda b, i: (b, 0, i)),
        out_shape=jax.ShapeDtypeStruct((B, _K, N1), jnp.int32),
    )(q_aug, k_aug)


def _sc_gather(table, xrow, yrow, zrow, idx):
    total = idx.shape[0]
    n_chunks = total // 128
    idx2d = idx.reshape(n_chunks, 128)
    per_w = n_chunks // 32
    npts = xrow.shape[0]
    mesh = plsc.VectorSubcoreMesh(core_axis_name="c", subcore_axis_name="s")

    @functools.partial(
        pl.kernel, mesh=mesh,
        compiler_params=pltpu.CompilerParams(needs_layout_passes=False),
        out_type=[
            jax.ShapeDtypeStruct((total, _C), jnp.float32),
            jax.ShapeDtypeStruct((n_chunks, 4, 128), jnp.float32),
        ],
        scratch_types=[
            pltpu.VMEM((128,), jnp.int32),
            pltpu.VMEM((128,), jnp.int32),
            pltpu.VMEM((128, _C), jnp.float32),
            pltpu.VMEM((128, _C), jnp.float32),
            pltpu.VMEM((4, 128), jnp.float32),
            pltpu.VMEM((4, 128), jnp.float32),
            pltpu.VMEM((npts,), jnp.float32),
            pltpu.VMEM((npts,), jnp.float32),
            pltpu.VMEM((npts,), jnp.float32),
            pltpu.SemaphoreType.DMA,
            pltpu.SemaphoreType.DMA,
        ],
    )
    def gk(table_hbm, x_hbm, y_hbm, z_hbm, idx_hbm, out_hbm, xyz_hbm,
           idxv0, idxv1, rows0, rows1, xyz0, xyz1, xv, yv, zv, sem0, sem1):
        wid = lax.axis_index("s") * 2 + lax.axis_index("c")
        base = wid * per_w
        pltpu.sync_copy(x_hbm, xv)
        pltpu.sync_copy(y_hbm, yv)
        pltpu.sync_copy(z_hbm, zv)

        def start(row, idxv, rows, sem):
            pltpu.sync_copy(idx_hbm.at[row], idxv)
            pltpu.async_copy(table_hbm.at[idxv], rows, sem)

        def finish(row, idxv, rows, xyzbuf, sem):
            pltpu.make_async_copy(table_hbm.at[idxv], rows, sem).wait()
            for g in range(8):
                iv = idxv[pl.ds(g * 16, 16)]
                xyzbuf[0, pl.ds(g * 16, 16)] = plsc.load_gather(xv, [iv])
                xyzbuf[1, pl.ds(g * 16, 16)] = plsc.load_gather(yv, [iv])
                xyzbuf[2, pl.ds(g * 16, 16)] = plsc.load_gather(zv, [iv])
            pltpu.sync_copy(rows, out_hbm.at[pl.ds(row * 128, 128)])
            pltpu.sync_copy(xyzbuf, xyz_hbm.at[row])

        start(base, idxv0, rows0, sem0)
        start(base + 1, idxv1, rows1, sem1)

        def body(i, carry):
            row = base + 2 * i
            finish(row, idxv0, rows0, xyz0, sem0)
            start(row + 2, idxv0, rows0, sem0)
            finish(row + 1, idxv1, rows1, xyz1, sem1)
            start(row + 3, idxv1, rows1, sem1)
            return carry

        lax.fori_loop(0, per_w // 2 - 1, body, 0)
        last = base + per_w - 2
        finish(last, idxv0, rows0, xyz0, sem0)
        finish(last + 1, idxv1, rows1, xyz1, sem1)

    return gk(table, xrow, yrow, zrow, idx2d)


def _phase3_body(g_ref, gx_ref, a1_ref, p1_ref, m1_ref, c0_ref, w1_ref,
                 b1_ref, v1_ref, c1_ref, v2_ref, c2_ref, out_ref):
    bs = p1_ref.shape[1]
    r = bs * _K
    a2 = g_ref[0].reshape(r, _C)
    xyzj = gx_ref[0].reshape(r, 4)[:, :3]
    p1 = p1_ref[0]
    p1r = jnp.broadcast_to(p1[None, :, :], (_K, bs, 3)).reshape(r, 3)
    d = xyzj - p1r
    t = _dot(d, m1_ref[...])
    h = jnp.maximum(t[:, :8] + c0_ref[...], 0.0)
    dirproj = t[:, 8:8 + _C]
    a1 = a1_ref[0]
    a1r = jnp.broadcast_to(a1[None, :, :], (_K, bs, _C)).reshape(r, _C)
    x = a1r + a2 + dirproj
    x = jnp.where(x >= 0.0, x, 0.1 * x)
    y = _dot(x, w1_ref[...]) + b1_ref[...]
    y = jnp.where(y >= 0.0, y, 0.1 * y)
    h = jnp.maximum(_dot(h, v1_ref[...]) + c1_ref[...], 0.0)
    w = jnp.maximum(_dot(h, v2_ref[...]) + c2_ref[...], 0.0)
    out_ref[0] = jnp.sum((w * y).reshape(_K, bs, _C), axis=0)


def _phase3(g1, g1x, a1rows, p1T, m1, c0, w1, b1, v1, c1, v2, c2):
    B, N1, _ = p1T.shape
    return pl.pallas_call(
        _phase3_body,
        grid=(B, N1 // _BS3),
        in_specs=[
            pl.BlockSpec((1, _K, _BS3, _C), lambda b, i: (b, 0, i, 0)),
            pl.BlockSpec((1, _K, _BS3, 4), lambda b, i: (b, 0, i, 0)),
            pl.BlockSpec((1, _BS3, _C), lambda b, i: (b, i, 0)),
            pl.BlockSpec((1, _BS3, 3), lambda b, i: (b, i, 0)),
            pl.BlockSpec((8 + _C, 3), lambda b, i: (0, 0)),
            pl.BlockSpec((1, 8), lambda b, i: (0, 0)),
            pl.BlockSpec((_C, _C), lambda b, i: (0, 0)),
            pl.BlockSpec((1, _C), lambda b, i: (0, 0)),
            pl.BlockSpec((8, 8), lambda b, i: (0, 0)),
            pl.BlockSpec((1, 8), lambda b, i: (0, 0)),
            pl.BlockSpec((_C, 8), lambda b, i: (0, 0)),
            pl.BlockSpec((1, _C), lambda b, i: (0, 0)),
        ],
        out_specs=pl.BlockSpec((1, _BS3, _C), lambda b, i: (b, i, 0)),
        out_shape=jax.ShapeDtypeStruct((B, N1, _C), jnp.float32),
    )(g1, g1x, a1rows, p1T, m1, c0, w1, b1, v1, c1, v2, c2)


def _phase5_body(g_ref, gx_ref, p1_ref, v0_ref, c0_ref, v1_ref, c1_ref,
                 v2_ref, c2_ref, out_ref):
    bs = p1_ref.shape[1]
    r = bs * _K
    nfj = g_ref[0].reshape(r, _C)
    xyzj = gx_ref[0].reshape(r, 4)[:, :3]
    p1 = p1_ref[0]
    p1r = jnp.broadcast_to(p1[None, :, :], (_K, bs, 3)).reshape(r, 3)
    d = xyzj - p1r
    h = jnp.maximum(_dot(d, v0_ref[...]) + c0_ref[...], 0.0)
    h = jnp.maximum(_dot(h, v1_ref[...]) + c1_ref[...], 0.0)
    w = jnp.maximum(_dot(h, v2_ref[...]) + c2_ref[...], 0.0)
    out_ref[0] = jnp.sum((w * nfj).reshape(_K, bs, _C), axis=0)


def _phase5(g2, g2x, p1T, v0, c0, v1, c1, v2, c2):
    B, N1, _ = p1T.shape
    return pl.pallas_call(
        _phase5_body,
        grid=(B, N1 // _BS3),
        in_specs=[
            pl.BlockSpec((1, _K, _BS3, _C), lambda b, i: (b, 0, i, 0)),
            pl.BlockSpec((1, _K, _BS3, 4), lambda b, i: (b, 0, i, 0)),
            pl.BlockSpec((1, _BS3, 3), lambda b, i: (b, i, 0)),
            pl.BlockSpec((8, 3), lambda b, i: (0, 0)),
            pl.BlockSpec((1, 8), lambda b, i: (0, 0)),
            pl.BlockSpec((8, 8), lambda b, i: (0, 0)),
            pl.BlockSpec((1, 8), lambda b, i: (0, 0)),
            pl.BlockSpec((_C, 8), lambda b, i: (0, 0)),
            pl.BlockSpec((1, _C), lambda b, i: (0, 0)),
        ],
        out_specs=pl.BlockSpec((1, _BS3, _C), lambda b, i: (b, i, 0)),
        out_shape=jax.ShapeDtypeStruct((B, N1, _C), jnp.float32),
    )(g2, g2x, p1T, v0, c0, v1, c1, v2, c2)


def kernel(pc1, pc2, feature1, feature2, mlp_W0, mlp_b0, mlp_W1, mlp_b1,
           wn1_W0, wn1_b0, wn1_W1, wn1_b1, wn1_W2, wn1_b2,
           wn2_W0, wn2_b0, wn2_W1, wn2_b1, wn2_W2, wn2_b2):
    B, _, N1 = pc1.shape
    N2 = pc2.shape[2]
    p1T = jnp.transpose(pc1, (0, 2, 1))
    w0a = mlp_W0[:, :_D]
    w0b = mlp_W0[:, _D:2 * _D]
    m1 = jnp.concatenate([wn1_W0, mlp_W0[:, 2 * _D:]], axis=0)

    p2T = jnp.transpose(pc2, (0, 2, 1))
    a1rows, t2, q1aug, k1aug, k2aug = _precompute(
        feature1, feature2, p1T, p2T, w0a, w0b, mlp_b0[None])
    total = B * N1 * _K
    p2rows = jnp.transpose(pc2, (1, 0, 2)).reshape(3, B * N2)
    p1rows = jnp.transpose(pc1, (1, 0, 2)).reshape(3, B * N1)

    idx1 = _knn(q1aug, k2aug)
    g1, g1xc = _sc_gather(t2.reshape(B * N2, _C),
                          p2rows[0], p2rows[1], p2rows[2], idx1.reshape(-1))
    idx2 = _knn(q1aug, k1aug)
    g1x = jnp.transpose(g1xc, (0, 2, 1)).reshape(total, 4)
    nf = _phase3(g1.reshape(B, _K, N1, _C), g1x.reshape(B, _K, N1, 4),
                 a1rows, p1T, m1,
                 wn1_b0[None], mlp_W1, mlp_b1[None],
                 wn1_W1, wn1_b1[None], wn1_W2, wn1_b2[None])

    g2, g2xc = _sc_gather(nf.reshape(B * N1, _C),
                          p1rows[0], p1rows[1], p1rows[2], idx2.reshape(-1))
    g2x = jnp.transpose(g2xc, (0, 2, 1)).reshape(total, 4)
    out_rows = _phase5(g2.reshape(B, _K, N1, _C), g2x.reshape(B, _K, N1, 4),
                       p1T,
                       wn2_W0, wn2_b0[None], wn2_W1, wn2_b1[None],
                       wn2_W2, wn2_b2[None])
    return jnp.transpose(out_rows, (0, 2, 1))
